# baseline (speedup 1.0000x reference)
import jax
import jax.numpy as jnp
from jax import lax
from jax.experimental import pallas as pl
from jax.experimental.pallas import tpu as pltpu
from jax.experimental.pallas import tpu_sc as plsc

N = 8192
D = 256
KN = 4096
L = 16
NW = 32
RPW = KN // NW
CR = 2
NCH = RPW // CR
HC = 32
NCV = KN // L
NDV = D // L


def _scores_body(h_ref, w_ref, b_ref, out_ref):
    acc = jnp.dot(h_ref[...], w_ref[...])
    out_ref[...] = jax.nn.sigmoid(acc + b_ref[0, 0])


def _scores(h, proj_w, proj_b):
    b2 = proj_b.reshape(1, 1)
    out = pl.pallas_call(
        _scores_body,
        out_shape=jax.ShapeDtypeStruct((N, 1), jnp.float32),
        in_specs=[
            pl.BlockSpec((N, D), lambda: (0, 0)),
            pl.BlockSpec((D, 1), lambda: (0, 0)),
            pl.BlockSpec(memory_space=pltpu.SMEM),
        ],
        out_specs=pl.BlockSpec((N, 1), lambda: (0, 0)),
    )(h, proj_w, b2)
    return out[:, 0]


def _sc_gather_body(g_hbm, h_hbm, idx_hbm, val_hbm,
                    gout_hbm, hout_hbm,
                    vidx, rowpad, vvals, grow, gout, hbuf,
                    gsem0, gsem1, osem00, osem01, osem10, osem11, hsem):
    cid = lax.axis_index("c")
    sid = lax.axis_index("s")
    wid = sid * 2 + cid
    base = wid * RPW

    pltpu.sync_copy(idx_hbm, vidx)
    pltpu.sync_copy(val_hbm.at[pl.ds(base, RPW)], vvals)

    for v in range(NCH * 8 // L):
        p = lax.iota(jnp.int32, L) + v * L
        src = (p >> 3) * CR + jnp.minimum(p & 7, CR - 1)
        rv = plsc.load_gather(vidx, [src + base])
        rowpad[pl.ds(v * L, L)] = rv

    gsems = (gsem0, gsem1)
    osems = ((osem00, osem01), (osem10, osem11))

    for b in range(2):
        pltpu.async_copy(g_hbm.at[rowpad.at[pl.ds(8 * b, CR)]],
                         grow.at[b], gsems[b])

    for hc in range(RPW // HC):
        pltpu.async_copy(h_hbm.at[vidx.at[pl.ds(base + hc * HC, HC)]],
                         hbuf, hsem).wait()

        def hrow(r, _):
            sval = plsc.load_gather(
                vvals, [jnp.full((L,), hc * HC, jnp.int32) + r])

            def hcv(cv, _):
                off = pl.multiple_of(cv * L, L)
                hbuf[r, pl.ds(off, L)] = hbuf[r, pl.ds(off, L)] * sval
                return 0

            return lax.fori_loop(0, NDV, hcv, 0)

        lax.fori_loop(0, HC, hrow, 0)
        pltpu.sync_copy(hbuf, hout_hbm.at[pl.ds(base + hc * HC, HC)])

    def chunk(c, b):
        pltpu.make_async_copy(g_hbm.at[pl.ds(0, CR)], grow.at[b],
                              gsems[b]).wait()
        for r in range(CR):
            zero = jnp.zeros((L,), jnp.float32)
            bidx = jnp.full((L,), b, jnp.int32)
            ridx = jnp.full((L,), r, jnp.int32)

            def cvbody(cv, acc):
                off = pl.multiple_of(cv * L, L)
                ci = vidx[pl.ds(off, L)]
                vals = plsc.load_gather(grow, [bidx, ridx, ci])
                gout[b, r, pl.ds(off, L)] = vals
                return acc + vals

            acc = lax.fori_loop(0, NCV, cvbody, zero)
            tot_v = jnp.full((L,), jnp.sum(acc), jnp.float32)
            scale = jnp.ones((L,), jnp.float32) / (tot_v + 1e-9)

            def scbody(cv, _):
                off = pl.multiple_of(cv * L, L)
                gout[b, r, pl.ds(off, L)] = gout[b, r, pl.ds(off, L)] * scale
                return 0

            @pl.when(c >= 2)
            def _():
                pltpu.make_async_copy(gout.at[b, r],
                                      gout_hbm.at[0], osems[b][r]).wait()

            lax.fori_loop(0, NCV, scbody, 0)
            rg = base + c * CR + r
            pltpu.async_copy(gout.at[b, r], gout_hbm.at[rg], osems[b][r])

        @pl.when(c + 2 < NCH)
        def _():
            off = pl.multiple_of(8 * (c + 2), 8)
            pltpu.async_copy(
                g_hbm.at[rowpad.at[pl.ds(off, CR)]],
                grow.at[b], gsems[b])

    def outer(i, _):
        for b in range(2):
            chunk(i * 2 + b, b)
        return 0

    lax.fori_loop(0, NCH // 2, outer, 0)

    for b in range(2):
        for r in range(CR):
            pltpu.make_async_copy(gout.at[b, r], gout_hbm.at[0],
                                  osems[b][r]).wait()


def _sc_gather(g, h, idx, values):
    mesh = plsc.VectorSubcoreMesh(core_axis_name="c", subcore_axis_name="s")
    f = pl.kernel(
        _sc_gather_body,
        out_type=(
            jax.ShapeDtypeStruct((KN, KN), jnp.float32),
            jax.ShapeDtypeStruct((KN, D), jnp.float32),
        ),
        mesh=mesh,
        compiler_params=pltpu.CompilerParams(needs_layout_passes=False),
        scratch_types=[
            pltpu.VMEM((KN,), jnp.int32),
            pltpu.VMEM((NCH * 8,), jnp.int32),
            pltpu.VMEM((RPW,), jnp.float32),
            pltpu.VMEM((2, CR, N), jnp.float32),
            pltpu.VMEM((2, CR, KN), jnp.float32),
            pltpu.VMEM((HC, D), jnp.float32),
            pltpu.SemaphoreType.DMA,
            pltpu.SemaphoreType.DMA,
            pltpu.SemaphoreType.DMA,
            pltpu.SemaphoreType.DMA,
            pltpu.SemaphoreType.DMA,
            pltpu.SemaphoreType.DMA,
            pltpu.SemaphoreType.DMA,
        ],
    )
    return f(g, h, idx, values)


def kernel(g, h, proj_w, proj_b):
    scores = _scores(h, proj_w, proj_b)
    values, idx = jax.lax.top_k(scores, KN)
    g_new, new_h = _sc_gather(g, h, idx, values)
    return g_new, new_h, idx


# speedup vs baseline: 3.3070x; 3.3070x over previous
import functools

import jax
import jax.numpy as jnp
from jax import lax
from jax.experimental import pallas as pl
from jax.experimental.pallas import tpu as pltpu
from jax.experimental.pallas import tpu_sc as plsc

N = 8192
D = 256
KN = 4096
L = 16
NW = 32
RPW = KN // NW
CR = 2
NCH = RPW // CR
HC = 32
NCV = KN // L
NDV = D // L


def _scores_body(h_ref, w_ref, b_ref, out_ref):
    acc = jnp.dot(h_ref[...], w_ref[...])
    out_ref[...] = jax.nn.sigmoid(acc + b_ref[0, 0])


def _scores(h, proj_w, proj_b):
    b2 = proj_b.reshape(1, 1)
    out = pl.pallas_call(
        _scores_body,
        out_shape=jax.ShapeDtypeStruct((N, 1), jnp.float32),
        in_specs=[
            pl.BlockSpec((N, D), lambda: (0, 0)),
            pl.BlockSpec((D, 1), lambda: (0, 0)),
            pl.BlockSpec(memory_space=pltpu.SMEM),
        ],
        out_specs=pl.BlockSpec((N, 1), lambda: (0, 0)),
    )(h, proj_w, b2)
    return out[:, 0]


NT = 16
EPT = N // NT
EV = EPT // L
NPASS = 4
OCH = KN // NT


def _sc_topk_body(scores_hbm, idx_hbm, val_hbm,
                  svals, keys, pay, histL, histT, allhist, tot, bbase, cnt,
                  ksc, psc, stg_k, stg_p, stg_pos, kout, pout, vbuf,
                  skeys_sh, spay_sh, hist_sh):
    cid = lax.axis_index("c")
    tid = lax.axis_index("s")
    base = tid * EPT
    iota = lax.iota(jnp.int32, L)
    zero_i = jnp.zeros((L,), jnp.int32)

    pltpu.sync_copy(scores_hbm.at[pl.ds(base, EPT)], svals)

    def mkkeys(v, _):
        off = pl.multiple_of(v * L, L)
        s = svals[pl.ds(off, L)]
        bu = lax.bitcast_convert_type(s, jnp.uint32)
        neg = (bu >> 31) == jnp.uint32(1)
        m = jnp.where(neg, ~bu, bu | jnp.uint32(0x80000000))
        keys[pl.ds(off, L)] = lax.bitcast_convert_type(~m, jnp.int32)
        pay[pl.ds(off, L)] = base + v * L + iota
        return 0

    lax.fori_loop(0, EV, mkkeys, 0)

    for p in range(NPASS):
        sh = 8 * p

        def zhist(b, _):
            off = pl.multiple_of(b * L, L)
            histL[pl.ds(off, L)] = zero_i
            return 0

        lax.fori_loop(0, 16 * 256 // L, zhist, 0)

        def dohist(v, _):
            off = pl.multiple_of(v * L, L)
            k = keys[pl.ds(off, L)]
            d = (lax.shift_right_logical(k, sh)) & 255
            plsc.addupdate_scatter(histL, [d * L + iota],
                                   jnp.ones((L,), jnp.int32))
            return 0

        lax.fori_loop(0, EV, dohist, 0)

        def redhist(b, _):
            off = pl.multiple_of(b * L, L)
            histL[pl.ds(off, L)] = plsc.cumsum(histL[pl.ds(off, L)])
            return 0

        lax.fori_loop(0, 256, redhist, 0)

        def gathist(bv, _):
            off = pl.multiple_of(bv * L, L)
            idxs = (bv * L + iota) * L + (L - 1)
            histT[pl.ds(off, L)] = plsc.load_gather(histL, [idxs])
            return 0

        lax.fori_loop(0, 256 // L, gathist, 0)

        pltpu.sync_copy(histT, hist_sh.at[tid])
        plsc.subcore_barrier()

        pltpu.sync_copy(hist_sh, allhist)

        def scan_bv(bv, _):
            off = pl.multiple_of(bv * L, L)

            def acc_t(t, carry):
                tv, mv = carry
                row = allhist[t, pl.ds(off, L)]
                tv = tv + row
                mv = mv + jnp.where(t < tid, row, zero_i)
                return (tv, mv)

            tv, mv = lax.fori_loop(0, NT, acc_t, (zero_i, zero_i))
            tot[pl.ds(off, L)] = tv
            bbase[pl.ds(off, L)] = mv
            return 0

        lax.fori_loop(0, 256 // L, scan_bv, 0)

        def excl_bv(bv, carry):
            off = pl.multiple_of(bv * L, L)
            tv = tot[pl.ds(off, L)]
            inc = plsc.cumsum(tv)
            bbase[pl.ds(off, L)] = bbase[pl.ds(off, L)] + (inc - tv) + carry
            return carry + jnp.sum(tv)

        lax.fori_loop(0, 256 // L, excl_bv, jnp.int32(0))

        def zcnt(bv, _):
            off = pl.multiple_of(bv * L, L)
            cnt[pl.ds(off, L)] = zero_i
            return 0

        lax.fori_loop(0, 256 // L, zcnt, 0)

        def permute(v, _):
            off = pl.multiple_of(v * L, L)
            k = keys[pl.ds(off, L)]
            pv = pay[pl.ds(off, L)]
            d = (lax.shift_right_logical(k, sh)) & 255
            d_s, lane_s = lax.sort([d, iota], dimension=0, is_stable=True,
                                   num_keys=1)
            ksc[...] = d_s
            prev = plsc.load_gather(ksc, [jnp.maximum(iota - 1, 0)])
            prev = jnp.where(iota == 0, jnp.full((L,), -1, jnp.int32), prev)
            seg = d_s != prev
            first = plsc.cummax(jnp.where(seg, iota, zero_i))
            subr = iota - first
            nxt = plsc.load_gather(ksc, [jnp.minimum(iota + 1, L - 1)])
            nxt = jnp.where(iota == L - 1, jnp.full((L,), -1, jnp.int32), nxt)
            last = d_s != nxt
            cnt_old = plsc.load_gather(cnt, [d_s])
            bb = plsc.load_gather(bbase, [d_s])
            pos = bb + cnt_old + subr
            plsc.store_scatter(cnt, [d_s], cnt_old + subr + 1, mask=last)
            ksc[...] = k
            psc[...] = pv
            k_s = plsc.load_gather(ksc, [lane_s])
            p_s = plsc.load_gather(psc, [lane_s])
            row = v >> 3
            col = pl.multiple_of((v & 7) * L, L)
            stg_k[row, pl.ds(col, L)] = k_s
            stg_p[row, pl.ds(col, L)] = p_s
            stg_pos[row, pl.ds(col, L)] = pos
            return 0

        lax.fori_loop(0, EV, permute, 0)

        for j in range(4):
            pltpu.sync_copy(stg_k.at[j], skeys_sh.at[stg_pos.at[j]])
            pltpu.sync_copy(stg_p.at[j], spay_sh.at[stg_pos.at[j]])
        plsc.subcore_barrier()

        if p < NPASS - 1:
            pltpu.sync_copy(skeys_sh.at[pl.ds(base, EPT)], keys)
            pltpu.sync_copy(spay_sh.at[pl.ds(base, EPT)], pay)

    @pl.when(cid == 0)
    def _():
        obase = tid * OCH
        pltpu.sync_copy(skeys_sh.at[pl.ds(obase, OCH)], kout)
        pltpu.sync_copy(spay_sh.at[pl.ds(obase, OCH)], pout)

        def unkey(ov, _):
            off = pl.multiple_of(ov * L, L)
            m = ~lax.bitcast_convert_type(kout[pl.ds(off, L)], jnp.uint32)
            negflag = (m >> 31) == jnp.uint32(1)
            bu = jnp.where(negflag, m & jnp.uint32(0x7FFFFFFF), ~m)
            vbuf[pl.ds(off, L)] = lax.bitcast_convert_type(bu, jnp.float32)
            return 0

        lax.fori_loop(0, OCH // L, unkey, 0)
        pltpu.sync_copy(pout, idx_hbm.at[pl.ds(obase, OCH)])
        pltpu.sync_copy(vbuf, val_hbm.at[pl.ds(obase, OCH)])


def _sc_topk(scores):
    mesh = plsc.VectorSubcoreMesh(core_axis_name="c", subcore_axis_name="s")
    f = pl.kernel(
        _sc_topk_body,
        out_type=(
            jax.ShapeDtypeStruct((KN,), jnp.int32),
            jax.ShapeDtypeStruct((KN,), jnp.float32),
        ),
        mesh=mesh,
        compiler_params=pltpu.CompilerParams(needs_layout_passes=False),
        scratch_types=[
            pltpu.VMEM((EPT,), jnp.float32),
            pltpu.VMEM((EPT,), jnp.int32),
            pltpu.VMEM((EPT,), jnp.int32),
            pltpu.VMEM((256 * L,), jnp.int32),
            pltpu.VMEM((256,), jnp.int32),
            pltpu.VMEM((NT, 256), jnp.int32),
            pltpu.VMEM((256,), jnp.int32),
            pltpu.VMEM((256,), jnp.int32),
            pltpu.VMEM((256,), jnp.int32),
            pltpu.VMEM((L,), jnp.int32),
            pltpu.VMEM((L,), jnp.int32),
            pltpu.VMEM((4, 128), jnp.int32),
            pltpu.VMEM((4, 128), jnp.int32),
            pltpu.VMEM((4, 128), jnp.int32),
            pltpu.VMEM((OCH,), jnp.int32),
            pltpu.VMEM((OCH,), jnp.int32),
            pltpu.VMEM((OCH,), jnp.float32),
            pltpu.VMEM_SHARED((N,), jnp.int32),
            pltpu.VMEM_SHARED((N,), jnp.int32),
            pltpu.VMEM_SHARED((NT, 256), jnp.int32),
        ],
    )
    return f(scores)


def _sc_gather_body(g_hbm, h_hbm, idx_hbm, val_hbm,
                    gout_hbm, hout_hbm,
                    vidx, rowpad, vvals, grow, gout, hbuf,
                    gsem0, gsem1, osem00, osem01, osem10, osem11, hsem):
    cid = lax.axis_index("c")
    sid = lax.axis_index("s")
    wid = sid * 2 + cid
    base = wid * RPW

    pltpu.sync_copy(idx_hbm, vidx)
    pltpu.sync_copy(val_hbm.at[pl.ds(base, RPW)], vvals)

    for v in range(NCH * 8 // L):
        p = lax.iota(jnp.int32, L) + v * L
        src = (p >> 3) * CR + jnp.minimum(p & 7, CR - 1)
        rv = plsc.load_gather(vidx, [src + base])
        rowpad[pl.ds(v * L, L)] = rv

    gsems = (gsem0, gsem1)
    osems = ((osem00, osem01), (osem10, osem11))

    for b in range(2):
        pltpu.async_copy(g_hbm.at[rowpad.at[pl.ds(8 * b, CR)]],
                         grow.at[b], gsems[b])

    for hc in range(RPW // HC):
        pltpu.async_copy(h_hbm.at[vidx.at[pl.ds(base + hc * HC, HC)]],
                         hbuf, hsem).wait()

        def hrow(r, _):
            sval = plsc.load_gather(
                vvals, [jnp.full((L,), hc * HC, jnp.int32) + r])

            def hcv(cv, _):
                off = pl.multiple_of(cv * L, L)
                hbuf[r, pl.ds(off, L)] = hbuf[r, pl.ds(off, L)] * sval
                return 0

            return lax.fori_loop(0, NDV, hcv, 0)

        lax.fori_loop(0, HC, hrow, 0)
        pltpu.sync_copy(hbuf, hout_hbm.at[pl.ds(base + hc * HC, HC)])

    def chunk(c, b):
        pltpu.make_async_copy(g_hbm.at[pl.ds(0, CR)], grow.at[b],
                              gsems[b]).wait()
        zero = jnp.zeros((L,), jnp.float32)
        bidx = jnp.full((L,), b, jnp.int32)
        r0i = jnp.full((L,), 0, jnp.int32)
        r1i = jnp.full((L,), 1, jnp.int32)

        @plsc.parallel_loop(0, NCV, 1, unroll=4, carry=(zero, zero))
        def accs(cv, carry):
            a0, a1 = carry
            off = pl.multiple_of(cv * L, L)
            ci = vidx[pl.ds(off, L)]
            v0 = plsc.load_gather(grow, [bidx, r0i, ci])
            v1 = plsc.load_gather(grow, [bidx, r1i, ci])
            gout[b, 0, pl.ds(off, L)] = v0
            gout[b, 1, pl.ds(off, L)] = v1
            return (a0 + v0, a1 + v1)

        for r in range(CR):
            tot_v = jnp.full((L,), jnp.sum(accs[r]), jnp.float32)
            scale = jnp.ones((L,), jnp.float32) / (tot_v + 1e-9)

            @pl.when(c >= 2)
            def _():
                pltpu.make_async_copy(gout.at[b, r],
                                      gout_hbm.at[0], osems[b][r]).wait()

            @plsc.parallel_loop(0, NCV, 1, unroll=4)
            def _(cv):
                off = pl.multiple_of(cv * L, L)
                gout[b, r, pl.ds(off, L)] = gout[b, r, pl.ds(off, L)] * scale

            rg = base + c * CR + r
            pltpu.async_copy(gout.at[b, r], gout_hbm.at[rg], osems[b][r])

        @pl.when(c + 2 < NCH)
        def _():
            off = pl.multiple_of(8 * (c + 2), 8)
            pltpu.async_copy(
                g_hbm.at[rowpad.at[pl.ds(off, CR)]],
                grow.at[b], gsems[b])

    def outer(i, _):
        for b in range(2):
            chunk(i * 2 + b, b)
        return 0

    lax.fori_loop(0, NCH // 2, outer, 0)

    for b in range(2):
        for r in range(CR):
            pltpu.make_async_copy(gout.at[b, r], gout_hbm.at[0],
                                  osems[b][r]).wait()


def _sc_gather(g, h, idx, values):
    mesh = plsc.VectorSubcoreMesh(core_axis_name="c", subcore_axis_name="s")
    f = pl.kernel(
        _sc_gather_body,
        out_type=(
            jax.ShapeDtypeStruct((KN, KN), jnp.float32),
            jax.ShapeDtypeStruct((KN, D), jnp.float32),
        ),
        mesh=mesh,
        compiler_params=pltpu.CompilerParams(needs_layout_passes=False),
        scratch_types=[
            pltpu.VMEM((KN,), jnp.int32),
            pltpu.VMEM((NCH * 8,), jnp.int32),
            pltpu.VMEM((RPW,), jnp.float32),
            pltpu.VMEM((2, CR, N), jnp.float32),
            pltpu.VMEM((2, CR, KN), jnp.float32),
            pltpu.VMEM((HC, D), jnp.float32),
            pltpu.SemaphoreType.DMA,
            pltpu.SemaphoreType.DMA,
            pltpu.SemaphoreType.DMA,
            pltpu.SemaphoreType.DMA,
            pltpu.SemaphoreType.DMA,
            pltpu.SemaphoreType.DMA,
            pltpu.SemaphoreType.DMA,
        ],
    )
    return f(g, h, idx, values)


def kernel(g, h, proj_w, proj_b):
    scores = _scores(h, proj_w, proj_b)
    idx, values = _sc_topk(scores)
    g_new, new_h = _sc_gather(g, h, idx, values)
    return g_new, new_h, idx


# speedup vs baseline: 3.4985x; 1.0579x over previous
import functools

import jax
import jax.numpy as jnp
from jax import lax
from jax.experimental import pallas as pl
from jax.experimental.pallas import tpu as pltpu
from jax.experimental.pallas import tpu_sc as plsc

N = 8192
D = 256
KN = 4096
L = 16
NW = 32
RPW = KN // NW
CR = 2
NCH = RPW // CR
HC = 32
NCV = KN // L
NDV = D // L


def _scores_body(h_ref, w_ref, b_ref, out_ref):
    acc = jnp.dot(h_ref[...], w_ref[...])
    out_ref[...] = jax.nn.sigmoid(acc + b_ref[0, 0])


def _scores(h, proj_w, proj_b):
    b2 = proj_b.reshape(1, 1)
    out = pl.pallas_call(
        _scores_body,
        out_shape=jax.ShapeDtypeStruct((N, 1), jnp.float32),
        in_specs=[
            pl.BlockSpec((N, D), lambda: (0, 0)),
            pl.BlockSpec((D, 1), lambda: (0, 0)),
            pl.BlockSpec(memory_space=pltpu.SMEM),
        ],
        out_specs=pl.BlockSpec((N, 1), lambda: (0, 0)),
    )(h, proj_w, b2)
    return out[:, 0]


NT = 16
EPT = N // NT
EV = EPT // L
NPASS = 4
OCH = KN // NT


def _sc_topk_body(scores_hbm, idx_hbm, val_hbm,
                  svals, keys, pay, histL, histT, allhist, tot, bbase, cnt,
                  ksc, psc, stg_k, stg_p, stg_pos, kout, pout, vbuf,
                  skeys_sh, spay_sh, hist_sh):
    cid = lax.axis_index("c")
    tid = lax.axis_index("s")
    base = tid * EPT
    iota = lax.iota(jnp.int32, L)
    zero_i = jnp.zeros((L,), jnp.int32)

    pltpu.sync_copy(scores_hbm.at[pl.ds(base, EPT)], svals)

    def mkkeys(v, _):
        off = pl.multiple_of(v * L, L)
        s = svals[pl.ds(off, L)]
        bu = lax.bitcast_convert_type(s, jnp.uint32)
        neg = (bu >> 31) == jnp.uint32(1)
        m = jnp.where(neg, ~bu, bu | jnp.uint32(0x80000000))
        keys[pl.ds(off, L)] = lax.bitcast_convert_type(~m, jnp.int32)
        pay[pl.ds(off, L)] = base + v * L + iota
        return 0

    lax.fori_loop(0, EV, mkkeys, 0)

    for p in range(NPASS):
        sh = 8 * p

        def zhist(b, _):
            off = pl.multiple_of(b * L, L)
            histL[pl.ds(off, L)] = zero_i
            return 0

        lax.fori_loop(0, 16 * 256 // L, zhist, 0)

        def dohist(v, _):
            off = pl.multiple_of(v * L, L)
            k = keys[pl.ds(off, L)]
            d = (lax.shift_right_logical(k, sh)) & 255
            plsc.addupdate_scatter(histL, [d * L + iota],
                                   jnp.ones((L,), jnp.int32))
            return 0

        lax.fori_loop(0, EV, dohist, 0)

        def redhist(b, _):
            off = pl.multiple_of(b * L, L)
            histL[pl.ds(off, L)] = plsc.cumsum(histL[pl.ds(off, L)])
            return 0

        lax.fori_loop(0, 256, redhist, 0)

        def gathist(bv, _):
            off = pl.multiple_of(bv * L, L)
            idxs = (bv * L + iota) * L + (L - 1)
            histT[pl.ds(off, L)] = plsc.load_gather(histL, [idxs])
            return 0

        lax.fori_loop(0, 256 // L, gathist, 0)

        pltpu.sync_copy(histT, hist_sh.at[tid])
        plsc.subcore_barrier()

        pltpu.sync_copy(hist_sh, allhist)

        def scan_bv(bv, _):
            off = pl.multiple_of(bv * L, L)

            def acc_t(t, carry):
                tv, mv = carry
                row = allhist[t, pl.ds(off, L)]
                tv = tv + row
                mv = mv + jnp.where(t < tid, row, zero_i)
                return (tv, mv)

            tv, mv = lax.fori_loop(0, NT, acc_t, (zero_i, zero_i))
            tot[pl.ds(off, L)] = tv
            bbase[pl.ds(off, L)] = mv
            return 0

        lax.fori_loop(0, 256 // L, scan_bv, 0)

        def excl_bv(bv, carry):
            off = pl.multiple_of(bv * L, L)
            tv = tot[pl.ds(off, L)]
            inc = plsc.cumsum(tv)
            bbase[pl.ds(off, L)] = bbase[pl.ds(off, L)] + (inc - tv) + carry
            return carry + jnp.sum(tv)

        lax.fori_loop(0, 256 // L, excl_bv, jnp.int32(0))

        def zcnt(bv, _):
            off = pl.multiple_of(bv * L, L)
            cnt[pl.ds(off, L)] = zero_i
            return 0

        lax.fori_loop(0, 256 // L, zcnt, 0)

        def permute(v, _):
            off = pl.multiple_of(v * L, L)
            k = keys[pl.ds(off, L)]
            pv = pay[pl.ds(off, L)]
            d = (lax.shift_right_logical(k, sh)) & 255
            d_s, lane_s = lax.sort([d, iota], dimension=0, is_stable=True,
                                   num_keys=1)
            ksc[...] = d_s
            prev = plsc.load_gather(ksc, [jnp.maximum(iota - 1, 0)])
            prev = jnp.where(iota == 0, jnp.full((L,), -1, jnp.int32), prev)
            seg = d_s != prev
            first = plsc.cummax(jnp.where(seg, iota, zero_i))
            subr = iota - first
            nxt = plsc.load_gather(ksc, [jnp.minimum(iota + 1, L - 1)])
            nxt = jnp.where(iota == L - 1, jnp.full((L,), -1, jnp.int32), nxt)
            last = d_s != nxt
            cnt_old = plsc.load_gather(cnt, [d_s])
            bb = plsc.load_gather(bbase, [d_s])
            pos = bb + cnt_old + subr
            plsc.store_scatter(cnt, [d_s], cnt_old + subr + 1, mask=last)
            ksc[...] = k
            psc[...] = pv
            k_s = plsc.load_gather(ksc, [lane_s])
            p_s = plsc.load_gather(psc, [lane_s])
            row = v >> 3
            col = pl.multiple_of((v & 7) * L, L)
            stg_k[row, pl.ds(col, L)] = k_s
            stg_p[row, pl.ds(col, L)] = p_s
            stg_pos[row, pl.ds(col, L)] = pos
            return 0

        lax.fori_loop(0, EV, permute, 0)

        for j in range(4):
            pltpu.sync_copy(stg_k.at[j], skeys_sh.at[stg_pos.at[j]])
            pltpu.sync_copy(stg_p.at[j], spay_sh.at[stg_pos.at[j]])
        plsc.subcore_barrier()

        if p < NPASS - 1:
            pltpu.sync_copy(skeys_sh.at[pl.ds(base, EPT)], keys)
            pltpu.sync_copy(spay_sh.at[pl.ds(base, EPT)], pay)

    @pl.when(cid == 0)
    def _():
        obase = tid * OCH
        pltpu.sync_copy(skeys_sh.at[pl.ds(obase, OCH)], kout)
        pltpu.sync_copy(spay_sh.at[pl.ds(obase, OCH)], pout)

        def unkey(ov, _):
            off = pl.multiple_of(ov * L, L)
            m = ~lax.bitcast_convert_type(kout[pl.ds(off, L)], jnp.uint32)
            negflag = (m >> 31) == jnp.uint32(1)
            bu = jnp.where(negflag, m & jnp.uint32(0x7FFFFFFF), ~m)
            vbuf[pl.ds(off, L)] = lax.bitcast_convert_type(bu, jnp.float32)
            return 0

        lax.fori_loop(0, OCH // L, unkey, 0)
        pltpu.sync_copy(pout, idx_hbm.at[pl.ds(obase, OCH)])
        pltpu.sync_copy(vbuf, val_hbm.at[pl.ds(obase, OCH)])


def _sc_topk(scores):
    mesh = plsc.VectorSubcoreMesh(core_axis_name="c", subcore_axis_name="s")
    f = pl.kernel(
        _sc_topk_body,
        out_type=(
            jax.ShapeDtypeStruct((KN,), jnp.int32),
            jax.ShapeDtypeStruct((KN,), jnp.float32),
        ),
        mesh=mesh,
        compiler_params=pltpu.CompilerParams(needs_layout_passes=False),
        scratch_types=[
            pltpu.VMEM((EPT,), jnp.float32),
            pltpu.VMEM((EPT,), jnp.int32),
            pltpu.VMEM((EPT,), jnp.int32),
            pltpu.VMEM((256 * L,), jnp.int32),
            pltpu.VMEM((256,), jnp.int32),
            pltpu.VMEM((NT, 256), jnp.int32),
            pltpu.VMEM((256,), jnp.int32),
            pltpu.VMEM((256,), jnp.int32),
            pltpu.VMEM((256,), jnp.int32),
            pltpu.VMEM((L,), jnp.int32),
            pltpu.VMEM((L,), jnp.int32),
            pltpu.VMEM((4, 128), jnp.int32),
            pltpu.VMEM((4, 128), jnp.int32),
            pltpu.VMEM((4, 128), jnp.int32),
            pltpu.VMEM((OCH,), jnp.int32),
            pltpu.VMEM((OCH,), jnp.int32),
            pltpu.VMEM((OCH,), jnp.float32),
            pltpu.VMEM_SHARED((N,), jnp.int32),
            pltpu.VMEM_SHARED((N,), jnp.int32),
            pltpu.VMEM_SHARED((NT, 256), jnp.int32),
        ],
    )
    return f(scores)


def _sc_gather_body(g_hbm, h_hbm, idx_hbm, val_hbm,
                    gout_hbm, hout_hbm,
                    vidx, rowpad, vvals, grow, gout, hbuf,
                    gsem0, gsem1, osem00, osem01, osem10, osem11, hsem):
    cid = lax.axis_index("c")
    sid = lax.axis_index("s")
    wid = sid * 2 + cid
    base = wid * RPW

    pltpu.sync_copy(idx_hbm, vidx)
    pltpu.sync_copy(val_hbm.at[pl.ds(base, RPW)], vvals)

    for v in range(NCH * 8 // L):
        p = lax.iota(jnp.int32, L) + v * L
        src = (p >> 3) * CR + jnp.minimum(p & 7, CR - 1)
        rv = plsc.load_gather(vidx, [src + base])
        rowpad[pl.ds(v * L, L)] = rv

    gsems = (gsem0, gsem1)
    osems = ((osem00, osem01), (osem10, osem11))

    for b in range(2):
        pltpu.async_copy(g_hbm.at[rowpad.at[pl.ds(8 * b, CR)]],
                         grow.at[b], gsems[b])

    for hc in range(RPW // HC):
        pltpu.async_copy(h_hbm.at[vidx.at[pl.ds(base + hc * HC, HC)]],
                         hbuf, hsem).wait()

        def hrow(r, _):
            sval = plsc.load_gather(
                vvals, [jnp.full((L,), hc * HC, jnp.int32) + r])

            def hcv(cv, _):
                off = pl.multiple_of(cv * L, L)
                hbuf[r, pl.ds(off, L)] = hbuf[r, pl.ds(off, L)] * sval
                return 0

            return lax.fori_loop(0, NDV, hcv, 0)

        lax.fori_loop(0, HC, hrow, 0)
        pltpu.sync_copy(hbuf, hout_hbm.at[pl.ds(base + hc * HC, HC)])

    def chunk(c, b):
        pltpu.make_async_copy(g_hbm.at[pl.ds(0, CR)], grow.at[b],
                              gsems[b]).wait()
        zero = jnp.zeros((L,), jnp.float32)
        bidx = jnp.full((L,), b, jnp.int32)
        r0i = jnp.full((L,), 0, jnp.int32)
        r1i = jnp.full((L,), 1, jnp.int32)

        @plsc.parallel_loop(0, NCV, 1, unroll=8, carry=(zero, zero))
        def accs(cv, carry):
            a0, a1 = carry
            off = pl.multiple_of(cv * L, L)
            ci = vidx[pl.ds(off, L)]
            v0 = plsc.load_gather(grow, [bidx, r0i, ci])
            v1 = plsc.load_gather(grow, [bidx, r1i, ci])
            gout[b, 0, pl.ds(off, L)] = v0
            gout[b, 1, pl.ds(off, L)] = v1
            return (a0 + v0, a1 + v1)

        for r in range(CR):
            tot_v = jnp.full((L,), jnp.sum(accs[r]), jnp.float32)
            scale = jnp.ones((L,), jnp.float32) / (tot_v + 1e-9)

            @pl.when(c >= 2)
            def _():
                pltpu.make_async_copy(gout.at[b, r],
                                      gout_hbm.at[0], osems[b][r]).wait()

            @plsc.parallel_loop(0, NCV, 1, unroll=8)
            def _(cv):
                off = pl.multiple_of(cv * L, L)
                gout[b, r, pl.ds(off, L)] = gout[b, r, pl.ds(off, L)] * scale

            rg = base + c * CR + r
            pltpu.async_copy(gout.at[b, r], gout_hbm.at[rg], osems[b][r])

        @pl.when(c + 2 < NCH)
        def _():
            off = pl.multiple_of(8 * (c + 2), 8)
            pltpu.async_copy(
                g_hbm.at[rowpad.at[pl.ds(off, CR)]],
                grow.at[b], gsems[b])

    def outer(i, _):
        for b in range(2):
            chunk(i * 2 + b, b)
        return 0

    lax.fori_loop(0, NCH // 2, outer, 0)

    for b in range(2):
        for r in range(CR):
            pltpu.make_async_copy(gout.at[b, r], gout_hbm.at[0],
                                  osems[b][r]).wait()


def _sc_gather(g, h, idx, values):
    mesh = plsc.VectorSubcoreMesh(core_axis_name="c", subcore_axis_name="s")
    f = pl.kernel(
        _sc_gather_body,
        out_type=(
            jax.ShapeDtypeStruct((KN, KN), jnp.float32),
            jax.ShapeDtypeStruct((KN, D), jnp.float32),
        ),
        mesh=mesh,
        compiler_params=pltpu.CompilerParams(needs_layout_passes=False),
        scratch_types=[
            pltpu.VMEM((KN,), jnp.int32),
            pltpu.VMEM((NCH * 8,), jnp.int32),
            pltpu.VMEM((RPW,), jnp.float32),
            pltpu.VMEM((2, CR, N), jnp.float32),
            pltpu.VMEM((2, CR, KN), jnp.float32),
            pltpu.VMEM((HC, D), jnp.float32),
            pltpu.SemaphoreType.DMA,
            pltpu.SemaphoreType.DMA,
            pltpu.SemaphoreType.DMA,
            pltpu.SemaphoreType.DMA,
            pltpu.SemaphoreType.DMA,
            pltpu.SemaphoreType.DMA,
            pltpu.SemaphoreType.DMA,
        ],
    )
    return f(g, h, idx, values)


def kernel(g, h, proj_w, proj_b):
    scores = _scores(h, proj_w, proj_b)
    idx, values = _sc_topk(scores)
    g_new, new_h = _sc_gather(g, h, idx, values)
    return g_new, new_h, idx


# speedup vs baseline: 3.6662x; 1.0479x over previous
import functools

import jax
import jax.numpy as jnp
from jax import lax
from jax.experimental import pallas as pl
from jax.experimental.pallas import tpu as pltpu
from jax.experimental.pallas import tpu_sc as plsc

N = 8192
D = 256
KN = 4096
L = 16
NW = 32
RPW = KN // NW
CR = 2
NCH = RPW // CR
HC = 32
NCV = KN // L
NDV = D // L


def _scores_body(h_ref, w_ref, b_ref, out_ref):
    acc = jnp.dot(h_ref[...], w_ref[...])
    out_ref[...] = jax.nn.sigmoid(acc + b_ref[0, 0])


def _scores(h, proj_w, proj_b):
    b2 = proj_b.reshape(1, 1)
    out = pl.pallas_call(
        _scores_body,
        out_shape=jax.ShapeDtypeStruct((N, 1), jnp.float32),
        in_specs=[
            pl.BlockSpec((N, D), lambda: (0, 0)),
            pl.BlockSpec((D, 1), lambda: (0, 0)),
            pl.BlockSpec(memory_space=pltpu.SMEM),
        ],
        out_specs=pl.BlockSpec((N, 1), lambda: (0, 0)),
    )(h, proj_w, b2)
    return out[:, 0]


NT = 16
EPT = N // NT
EV = EPT // L
NPASS = 4
OCH = KN // NT


def _sc_fused_body(scores_hbm, g_hbm, h_hbm,
                   gout_hbm, hout_hbm, idx_hbm,
                   svals, keys, pay, histL, histT, allhist, tot, bbase, cnt,
                   ksc, psc, stg_k, stg_p, stg_pos, pout, kb,
                   vidx, rowpad, vvals, grow, gout, hbuf,
                   gsem0, gsem1, osem00, osem01, osem10, osem11, hsem,
                   skeys_sh, spay_sh, hist_sh):
    cid = lax.axis_index("c")
    tid = lax.axis_index("s")
    base = tid * EPT
    iota = lax.iota(jnp.int32, L)
    zero_i = jnp.zeros((L,), jnp.int32)

    pltpu.sync_copy(scores_hbm.at[pl.ds(base, EPT)], svals)

    def mkkeys(v, _):
        off = pl.multiple_of(v * L, L)
        s = svals[pl.ds(off, L)]
        bu = lax.bitcast_convert_type(s, jnp.uint32)
        neg = (bu >> 31) == jnp.uint32(1)
        m = jnp.where(neg, ~bu, bu | jnp.uint32(0x80000000))
        keys[pl.ds(off, L)] = lax.bitcast_convert_type(~m, jnp.int32)
        pay[pl.ds(off, L)] = base + v * L + iota
        return 0

    lax.fori_loop(0, EV, mkkeys, 0)

    for p in range(NPASS):
        sh = 8 * p

        def zhist(b, _):
            off = pl.multiple_of(b * L, L)
            histL[pl.ds(off, L)] = zero_i
            return 0

        lax.fori_loop(0, 16 * 256 // L, zhist, 0)

        def dohist(v, _):
            off = pl.multiple_of(v * L, L)
            k = keys[pl.ds(off, L)]
            d = (lax.shift_right_logical(k, sh)) & 255
            plsc.addupdate_scatter(histL, [d * L + iota],
                                   jnp.ones((L,), jnp.int32))
            return 0

        lax.fori_loop(0, EV, dohist, 0)

        def redhist(b, _):
            off = pl.multiple_of(b * L, L)
            histL[pl.ds(off, L)] = plsc.cumsum(histL[pl.ds(off, L)])
            return 0

        lax.fori_loop(0, 256, redhist, 0)

        def gathist(bv, _):
            off = pl.multiple_of(bv * L, L)
            idxs = (bv * L + iota) * L + (L - 1)
            histT[pl.ds(off, L)] = plsc.load_gather(histL, [idxs])
            return 0

        lax.fori_loop(0, 256 // L, gathist, 0)

        pltpu.sync_copy(histT, hist_sh.at[tid])
        plsc.subcore_barrier()

        pltpu.sync_copy(hist_sh, allhist)

        def scan_bv(bv, _):
            off = pl.multiple_of(bv * L, L)

            def acc_t(t, carry):
                tv, mv = carry
                row = allhist[t, pl.ds(off, L)]
                tv = tv + row
                mv = mv + jnp.where(t < tid, row, zero_i)
                return (tv, mv)

            tv, mv = lax.fori_loop(0, NT, acc_t, (zero_i, zero_i))
            tot[pl.ds(off, L)] = tv
            bbase[pl.ds(off, L)] = mv
            return 0

        lax.fori_loop(0, 256 // L, scan_bv, 0)

        def excl_bv(bv, carry):
            off = pl.multiple_of(bv * L, L)
            tv = tot[pl.ds(off, L)]
            inc = plsc.cumsum(tv)
            bbase[pl.ds(off, L)] = bbase[pl.ds(off, L)] + (inc - tv) + carry
            return carry + jnp.sum(tv)

        lax.fori_loop(0, 256 // L, excl_bv, jnp.int32(0))

        def zcnt(bv, _):
            off = pl.multiple_of(bv * L, L)
            cnt[pl.ds(off, L)] = zero_i
            return 0

        lax.fori_loop(0, 256 // L, zcnt, 0)

        def permute(v, _):
            off = pl.multiple_of(v * L, L)
            k = keys[pl.ds(off, L)]
            pv = pay[pl.ds(off, L)]
            d = (lax.shift_right_logical(k, sh)) & 255
            d_s, lane_s = lax.sort([d, iota], dimension=0, is_stable=True,
                                   num_keys=1)
            ksc[...] = d_s
            prev = plsc.load_gather(ksc, [jnp.maximum(iota - 1, 0)])
            prev = jnp.where(iota == 0, jnp.full((L,), -1, jnp.int32), prev)
            seg = d_s != prev
            first = plsc.cummax(jnp.where(seg, iota, zero_i))
            subr = iota - first
            nxt = plsc.load_gather(ksc, [jnp.minimum(iota + 1, L - 1)])
            nxt = jnp.where(iota == L - 1, jnp.full((L,), -1, jnp.int32), nxt)
            last = d_s != nxt
            cnt_old = plsc.load_gather(cnt, [d_s])
            bb = plsc.load_gather(bbase, [d_s])
            pos = bb + cnt_old + subr
            plsc.store_scatter(cnt, [d_s], cnt_old + subr + 1, mask=last)
            ksc[...] = k
            psc[...] = pv
            k_s = plsc.load_gather(ksc, [lane_s])
            p_s = plsc.load_gather(psc, [lane_s])
            row = v >> 3
            col = pl.multiple_of((v & 7) * L, L)
            stg_k[row, pl.ds(col, L)] = k_s
            stg_p[row, pl.ds(col, L)] = p_s
            stg_pos[row, pl.ds(col, L)] = pos
            return 0

        lax.fori_loop(0, EV, permute, 0)

        for j in range(4):
            pltpu.sync_copy(stg_k.at[j], skeys_sh.at[stg_pos.at[j]])
            pltpu.sync_copy(stg_p.at[j], spay_sh.at[stg_pos.at[j]])
        plsc.subcore_barrier()

        if p < NPASS - 1:
            pltpu.sync_copy(skeys_sh.at[pl.ds(base, EPT)], keys)
            pltpu.sync_copy(spay_sh.at[pl.ds(base, EPT)], pay)

    @pl.when(cid == 0)
    def _():
        obase = tid * OCH
        pltpu.sync_copy(spay_sh.at[pl.ds(obase, OCH)], pout)
        pltpu.sync_copy(pout, idx_hbm.at[pl.ds(obase, OCH)])

    sid = tid
    wid = sid * 2 + cid
    base = wid * RPW

    pltpu.sync_copy(spay_sh.at[pl.ds(0, KN)], vidx)
    pltpu.sync_copy(skeys_sh.at[pl.ds(base, RPW)], kb)

    def unkey(ov, _):
        off = pl.multiple_of(ov * L, L)
        m = ~lax.bitcast_convert_type(kb[pl.ds(off, L)], jnp.uint32)
        negflag = (m >> 31) == jnp.uint32(1)
        bu = jnp.where(negflag, m & jnp.uint32(0x7FFFFFFF), ~m)
        vvals[pl.ds(off, L)] = lax.bitcast_convert_type(bu, jnp.float32)
        return 0

    lax.fori_loop(0, RPW // L, unkey, 0)

    for v in range(NCH * 8 // L):
        p = lax.iota(jnp.int32, L) + v * L
        src = (p >> 3) * CR + jnp.minimum(p & 7, CR - 1)
        rv = plsc.load_gather(vidx, [src + base])
        rowpad[pl.ds(v * L, L)] = rv

    gsems = (gsem0, gsem1)
    osems = ((osem00, osem01), (osem10, osem11))

    for b in range(2):
        pltpu.async_copy(g_hbm.at[rowpad.at[pl.ds(8 * b, CR)]],
                         grow.at[b], gsems[b])

    for hc in range(RPW // HC):
        pltpu.async_copy(h_hbm.at[vidx.at[pl.ds(base + hc * HC, HC)]],
                         hbuf, hsem).wait()

        def hrow(r, _):
            sval = plsc.load_gather(
                vvals, [jnp.full((L,), hc * HC, jnp.int32) + r])

            def hcv(cv, _):
                off = pl.multiple_of(cv * L, L)
                hbuf[r, pl.ds(off, L)] = hbuf[r, pl.ds(off, L)] * sval
                return 0

            return lax.fori_loop(0, NDV, hcv, 0)

        lax.fori_loop(0, HC, hrow, 0)
        pltpu.sync_copy(hbuf, hout_hbm.at[pl.ds(base + hc * HC, HC)])

    def chunk(c, b):
        pltpu.make_async_copy(g_hbm.at[pl.ds(0, CR)], grow.at[b],
                              gsems[b]).wait()
        zero = jnp.zeros((L,), jnp.float32)
        bidx = jnp.full((L,), b, jnp.int32)
        r0i = jnp.full((L,), 0, jnp.int32)
        r1i = jnp.full((L,), 1, jnp.int32)

        @plsc.parallel_loop(0, NCV, 1, unroll=8, carry=(zero, zero))
        def accs(cv, carry):
            a0, a1 = carry
            off = pl.multiple_of(cv * L, L)
            ci = vidx[pl.ds(off, L)]
            v0 = plsc.load_gather(grow, [bidx, r0i, ci])
            v1 = plsc.load_gather(grow, [bidx, r1i, ci])
            gout[b, 0, pl.ds(off, L)] = v0
            gout[b, 1, pl.ds(off, L)] = v1
            return (a0 + v0, a1 + v1)

        for r in range(CR):
            tot_v = jnp.full((L,), jnp.sum(accs[r]), jnp.float32)
            scale = jnp.ones((L,), jnp.float32) / (tot_v + 1e-9)

            @pl.when(c >= 2)
            def _():
                pltpu.make_async_copy(gout.at[b, r],
                                      gout_hbm.at[0], osems[b][r]).wait()

            @plsc.parallel_loop(0, NCV, 1, unroll=8)
            def _(cv):
                off = pl.multiple_of(cv * L, L)
                gout[b, r, pl.ds(off, L)] = gout[b, r, pl.ds(off, L)] * scale

            rg = base + c * CR + r
            pltpu.async_copy(gout.at[b, r], gout_hbm.at[rg], osems[b][r])

        @pl.when(c + 2 < NCH)
        def _():
            off = pl.multiple_of(8 * (c + 2), 8)
            pltpu.async_copy(
                g_hbm.at[rowpad.at[pl.ds(off, CR)]],
                grow.at[b], gsems[b])

    def outer(i, _):
        for b in range(2):
            chunk(i * 2 + b, b)
        return 0

    lax.fori_loop(0, NCH // 2, outer, 0)

    for b in range(2):
        for r in range(CR):
            pltpu.make_async_copy(gout.at[b, r], gout_hbm.at[0],
                                  osems[b][r]).wait()


def _sc_fused(scores, g, h):
    mesh = plsc.VectorSubcoreMesh(core_axis_name="c", subcore_axis_name="s")
    f = pl.kernel(
        _sc_fused_body,
        out_type=(
            jax.ShapeDtypeStruct((KN, KN), jnp.float32),
            jax.ShapeDtypeStruct((KN, D), jnp.float32),
            jax.ShapeDtypeStruct((KN,), jnp.int32),
        ),
        mesh=mesh,
        compiler_params=pltpu.CompilerParams(needs_layout_passes=False),
        scratch_types=[
            pltpu.VMEM((EPT,), jnp.float32),
            pltpu.VMEM((EPT,), jnp.int32),
            pltpu.VMEM((EPT,), jnp.int32),
            pltpu.VMEM((256 * L,), jnp.int32),
            pltpu.VMEM((256,), jnp.int32),
            pltpu.VMEM((NT, 256), jnp.int32),
            pltpu.VMEM((256,), jnp.int32),
            pltpu.VMEM((256,), jnp.int32),
            pltpu.VMEM((256,), jnp.int32),
            pltpu.VMEM((L,), jnp.int32),
            pltpu.VMEM((L,), jnp.int32),
            pltpu.VMEM((4, 128), jnp.int32),
            pltpu.VMEM((4, 128), jnp.int32),
            pltpu.VMEM((4, 128), jnp.int32),
            pltpu.VMEM((OCH,), jnp.int32),
            pltpu.VMEM((RPW,), jnp.int32),
            pltpu.VMEM((KN,), jnp.int32),
            pltpu.VMEM((NCH * 8,), jnp.int32),
            pltpu.VMEM((RPW,), jnp.float32),
            pltpu.VMEM((2, CR, N), jnp.float32),
            pltpu.VMEM((2, CR, KN), jnp.float32),
            pltpu.VMEM((HC, D), jnp.float32),
            pltpu.SemaphoreType.DMA,
            pltpu.SemaphoreType.DMA,
            pltpu.SemaphoreType.DMA,
            pltpu.SemaphoreType.DMA,
            pltpu.SemaphoreType.DMA,
            pltpu.SemaphoreType.DMA,
            pltpu.SemaphoreType.DMA,
            pltpu.VMEM_SHARED((N,), jnp.int32),
            pltpu.VMEM_SHARED((N,), jnp.int32),
            pltpu.VMEM_SHARED((NT, 256), jnp.int32),
        ],
    )
    return f(scores, g, h)


def kernel(g, h, proj_w, proj_b):
    scores = _scores(h, proj_w, proj_b)
    g_new, new_h, idx = _sc_fused(scores, g, h)
    return g_new, new_h, idx


# speedup vs baseline: 3.7918x; 1.0343x over previous
import functools

import jax
import jax.numpy as jnp
from jax import lax
from jax.experimental import pallas as pl
from jax.experimental.pallas import tpu as pltpu
from jax.experimental.pallas import tpu_sc as plsc

N = 8192
D = 256
KN = 4096
L = 16
NW = 32
RPW = KN // NW
CR = 2
NCH = RPW // CR
HC = 32
NCV = KN // L
NDV = D // L


def _scores_body(h_ref, w_ref, b_ref, out_ref):
    acc = jnp.dot(h_ref[...], w_ref[...])
    out_ref[...] = jax.nn.sigmoid(acc + b_ref[0, 0])


def _scores(h, proj_w, proj_b):
    b2 = proj_b.reshape(1, 1)
    out = pl.pallas_call(
        _scores_body,
        out_shape=jax.ShapeDtypeStruct((N, 1), jnp.float32),
        in_specs=[
            pl.BlockSpec((N, D), lambda: (0, 0)),
            pl.BlockSpec((D, 1), lambda: (0, 0)),
            pl.BlockSpec(memory_space=pltpu.SMEM),
        ],
        out_specs=pl.BlockSpec((N, 1), lambda: (0, 0)),
    )(h, proj_w, b2)
    return out[:, 0]


NT = 16
EPT = N // NT
EV = EPT // L
NPASS = 4
OCH = KN // NT


def _sc_fused_body(scores_hbm, g_hbm, h_hbm,
                   gout_hbm, hout_hbm, idx_hbm,
                   svals, keys, pay, histL, histT, allhist, tot, bbase, cnt,
                   ksc, psc, stg_k, stg_p, stg_pos, pout, kb,
                   vidx, rowpad, vvals, grow, gout, hbuf,
                   gsem0, gsem1, osem00, osem01, osem10, osem11, hsem,
                   skeys_sh, spay_sh, hist_sh):
    cid = lax.axis_index("c")
    tid = lax.axis_index("s")
    base = tid * EPT
    iota = lax.iota(jnp.int32, L)
    zero_i = jnp.zeros((L,), jnp.int32)

    pltpu.sync_copy(scores_hbm.at[pl.ds(base, EPT)], svals)

    def mkkeys(v, _):
        off = pl.multiple_of(v * L, L)
        s = svals[pl.ds(off, L)]
        bu = lax.bitcast_convert_type(s, jnp.uint32)
        neg = (bu >> 31) == jnp.uint32(1)
        m = jnp.where(neg, ~bu, bu | jnp.uint32(0x80000000))
        keys[pl.ds(off, L)] = lax.bitcast_convert_type(~m, jnp.int32)
        pay[pl.ds(off, L)] = base + v * L + iota
        return 0

    lax.fori_loop(0, EV, mkkeys, 0)

    for p in range(NPASS):
        sh = 8 * p

        def zhist(b, _):
            off = pl.multiple_of(b * L, L)
            histL[pl.ds(off, L)] = zero_i
            return 0

        lax.fori_loop(0, 16 * 256 // L, zhist, 0)

        def dohist(v, _):
            off = pl.multiple_of(v * L, L)
            k = keys[pl.ds(off, L)]
            d = (lax.shift_right_logical(k, sh)) & 255
            plsc.addupdate_scatter(histL, [d * L + iota],
                                   jnp.ones((L,), jnp.int32))
            return 0

        lax.fori_loop(0, EV, dohist, 0)

        def redhist(b, _):
            off = pl.multiple_of(b * L, L)
            histL[pl.ds(off, L)] = plsc.cumsum(histL[pl.ds(off, L)])
            return 0

        lax.fori_loop(0, 256, redhist, 0)

        def gathist(bv, _):
            off = pl.multiple_of(bv * L, L)
            idxs = (bv * L + iota) * L + (L - 1)
            histT[pl.ds(off, L)] = plsc.load_gather(histL, [idxs])
            return 0

        lax.fori_loop(0, 256 // L, gathist, 0)

        pltpu.sync_copy(histT, hist_sh.at[tid])
        plsc.subcore_barrier()

        pltpu.sync_copy(hist_sh, allhist)

        def scan_bv(bv, _):
            off = pl.multiple_of(bv * L, L)

            def acc_t(t, carry):
                tv, mv = carry
                row = allhist[t, pl.ds(off, L)]
                tv = tv + row
                mv = mv + jnp.where(t < tid, row, zero_i)
                return (tv, mv)

            tv, mv = lax.fori_loop(0, NT, acc_t, (zero_i, zero_i))
            tot[pl.ds(off, L)] = tv
            bbase[pl.ds(off, L)] = mv
            return 0

        lax.fori_loop(0, 256 // L, scan_bv, 0)

        def excl_bv(bv, carry):
            off = pl.multiple_of(bv * L, L)
            tv = tot[pl.ds(off, L)]
            inc = plsc.cumsum(tv)
            bbase[pl.ds(off, L)] = bbase[pl.ds(off, L)] + (inc - tv) + carry
            return carry + jnp.sum(tv)

        lax.fori_loop(0, 256 // L, excl_bv, jnp.int32(0))

        def zcnt(bv, _):
            off = pl.multiple_of(bv * L, L)
            cnt[pl.ds(off, L)] = zero_i
            return 0

        lax.fori_loop(0, 256 // L, zcnt, 0)

        def permute(v, _):
            off = pl.multiple_of(v * L, L)
            k = keys[pl.ds(off, L)]
            pv = pay[pl.ds(off, L)]
            d = (lax.shift_right_logical(k, sh)) & 255
            d_s, lane_s = lax.sort([d, iota], dimension=0, is_stable=True,
                                   num_keys=1)
            ksc[...] = d_s
            prev = plsc.load_gather(ksc, [jnp.maximum(iota - 1, 0)])
            prev = jnp.where(iota == 0, jnp.full((L,), -1, jnp.int32), prev)
            seg = d_s != prev
            first = plsc.cummax(jnp.where(seg, iota, zero_i))
            subr = iota - first
            nxt = plsc.load_gather(ksc, [jnp.minimum(iota + 1, L - 1)])
            nxt = jnp.where(iota == L - 1, jnp.full((L,), -1, jnp.int32), nxt)
            last = d_s != nxt
            cnt_old = plsc.load_gather(cnt, [d_s])
            bb = plsc.load_gather(bbase, [d_s])
            pos = bb + cnt_old + subr
            plsc.store_scatter(cnt, [d_s], cnt_old + subr + 1, mask=last)
            ksc[...] = k
            psc[...] = pv
            k_s = plsc.load_gather(ksc, [lane_s])
            p_s = plsc.load_gather(psc, [lane_s])
            row = v >> 3
            col = pl.multiple_of((v & 7) * L, L)
            stg_k[row, pl.ds(col, L)] = k_s
            stg_p[row, pl.ds(col, L)] = p_s
            stg_pos[row, pl.ds(col, L)] = pos
            return 0

        lax.fori_loop(0, EV, permute, 0)

        for j in range(4):
            pltpu.sync_copy(stg_k.at[j], skeys_sh.at[stg_pos.at[j]])
            pltpu.sync_copy(stg_p.at[j], spay_sh.at[stg_pos.at[j]])
        plsc.subcore_barrier()

        if p < NPASS - 1:
            pltpu.sync_copy(skeys_sh.at[pl.ds(base, EPT)], keys)
            pltpu.sync_copy(spay_sh.at[pl.ds(base, EPT)], pay)

    @pl.when(cid == 0)
    def _():
        obase = tid * OCH
        pltpu.sync_copy(spay_sh.at[pl.ds(obase, OCH)], pout)
        pltpu.sync_copy(pout, idx_hbm.at[pl.ds(obase, OCH)])

    sid = tid
    wid = sid * 2 + cid
    base = wid * RPW

    pltpu.sync_copy(spay_sh.at[pl.ds(0, KN)], vidx)
    pltpu.sync_copy(skeys_sh.at[pl.ds(base, RPW)], kb)

    def unkey(ov, _):
        off = pl.multiple_of(ov * L, L)
        m = ~lax.bitcast_convert_type(kb[pl.ds(off, L)], jnp.uint32)
        negflag = (m >> 31) == jnp.uint32(1)
        bu = jnp.where(negflag, m & jnp.uint32(0x7FFFFFFF), ~m)
        vvals[pl.ds(off, L)] = lax.bitcast_convert_type(bu, jnp.float32)
        return 0

    lax.fori_loop(0, RPW // L, unkey, 0)

    for v in range(NCH * 8 // L):
        p = lax.iota(jnp.int32, L) + v * L
        src = (p >> 3) * CR + jnp.minimum(p & 7, CR - 1)
        rv = plsc.load_gather(vidx, [src + base])
        rowpad[pl.ds(v * L, L)] = rv

    gsems = (gsem0, gsem1)
    osems = ((osem00, osem01), (osem10, osem11))

    for b in range(2):
        pltpu.async_copy(g_hbm.at[rowpad.at[pl.ds(8 * b, CR)]],
                         grow.at[b], gsems[b])

    for hc in range(RPW // HC):
        pltpu.async_copy(h_hbm.at[vidx.at[pl.ds(base + hc * HC, HC)]],
                         hbuf, hsem).wait()

        def hrow(r, _):
            sval = plsc.load_gather(
                vvals, [jnp.full((L,), hc * HC, jnp.int32) + r])

            def hcv(cv, _):
                off = pl.multiple_of(cv * L, L)
                hbuf[r, pl.ds(off, L)] = hbuf[r, pl.ds(off, L)] * sval
                return 0

            return lax.fori_loop(0, NDV, hcv, 0)

        lax.fori_loop(0, HC, hrow, 0)
        pltpu.sync_copy(hbuf, hout_hbm.at[pl.ds(base + hc * HC, HC)])

    def chunk(c, b):
        pltpu.make_async_copy(g_hbm.at[pl.ds(0, CR)], grow.at[b],
                              gsems[b]).wait()
        zero = jnp.zeros((L,), jnp.float32)
        bidx = jnp.full((L,), b, jnp.int32)
        r0i = jnp.full((L,), 0, jnp.int32)
        r1i = jnp.full((L,), 1, jnp.int32)

        gb = (b * CR + 0) * KN
        gb1 = (b * CR + 1) * KN

        @plsc.parallel_loop(0, NCV, 1, unroll=8, carry=(zero, zero))
        def accs(cv, carry):
            a0, a1 = carry
            off = pl.multiple_of(cv * L, L)
            ci = vidx[pl.ds(off, L)]
            v0 = plsc.load_gather(grow, [bidx, r0i, ci])
            v1 = plsc.load_gather(grow, [bidx, r1i, ci])
            gout[pl.ds(gb + off, L)] = v0
            gout[pl.ds(gb1 + off, L)] = v1
            return (a0 + v0, a1 + v1)

        scales = []
        for r in range(CR):
            tot_v = jnp.full((L,), jnp.sum(accs[r]), jnp.float32)
            scales.append(jnp.ones((L,), jnp.float32) / (tot_v + 1e-9))

            @pl.when(c >= 2)
            def _():
                pltpu.make_async_copy(gout.at[pl.ds(0, KN)],
                                      gout_hbm.at[0], osems[b][r]).wait()

        @plsc.parallel_loop(0, NCV, 1, unroll=8)
        def _(cv):
            off = pl.multiple_of(cv * L, L)
            gout[pl.ds(gb + off, L)] = gout[pl.ds(gb + off, L)] * scales[0]
            gout[pl.ds(gb1 + off, L)] = gout[pl.ds(gb1 + off, L)] * scales[1]

        for r in range(CR):
            rg = base + c * CR + r
            pltpu.async_copy(gout.at[pl.ds((b * CR + r) * KN, KN)],
                             gout_hbm.at[rg], osems[b][r])

        @pl.when(c + 2 < NCH)
        def _():
            off = pl.multiple_of(8 * (c + 2), 8)
            pltpu.async_copy(
                g_hbm.at[rowpad.at[pl.ds(off, CR)]],
                grow.at[b], gsems[b])

    def outer(i, _):
        for b in range(2):
            chunk(i * 2 + b, b)
        return 0

    lax.fori_loop(0, NCH // 2, outer, 0)

    for b in range(2):
        for r in range(CR):
            pltpu.make_async_copy(gout.at[pl.ds(0, KN)], gout_hbm.at[0],
                                  osems[b][r]).wait()


def _sc_fused(scores, g, h):
    mesh = plsc.VectorSubcoreMesh(core_axis_name="c", subcore_axis_name="s")
    f = pl.kernel(
        _sc_fused_body,
        out_type=(
            jax.ShapeDtypeStruct((KN, KN), jnp.float32),
            jax.ShapeDtypeStruct((KN, D), jnp.float32),
            jax.ShapeDtypeStruct((KN,), jnp.int32),
        ),
        mesh=mesh,
        compiler_params=pltpu.CompilerParams(needs_layout_passes=False),
        scratch_types=[
            pltpu.VMEM((EPT,), jnp.float32),
            pltpu.VMEM((EPT,), jnp.int32),
            pltpu.VMEM((EPT,), jnp.int32),
            pltpu.VMEM((256 * L,), jnp.int32),
            pltpu.VMEM((256,), jnp.int32),
            pltpu.VMEM((NT, 256), jnp.int32),
            pltpu.VMEM((256,), jnp.int32),
            pltpu.VMEM((256,), jnp.int32),
            pltpu.VMEM((256,), jnp.int32),
            pltpu.VMEM((L,), jnp.int32),
            pltpu.VMEM((L,), jnp.int32),
            pltpu.VMEM((4, 128), jnp.int32),
            pltpu.VMEM((4, 128), jnp.int32),
            pltpu.VMEM((4, 128), jnp.int32),
            pltpu.VMEM((OCH,), jnp.int32),
            pltpu.VMEM((RPW,), jnp.int32),
            pltpu.VMEM((KN,), jnp.int32),
            pltpu.VMEM((NCH * 8,), jnp.int32),
            pltpu.VMEM((RPW,), jnp.float32),
            pltpu.VMEM((2, CR, N), jnp.float32),
            pltpu.VMEM((2 * CR * KN,), jnp.float32),
            pltpu.VMEM((HC, D), jnp.float32),
            pltpu.SemaphoreType.DMA,
            pltpu.SemaphoreType.DMA,
            pltpu.SemaphoreType.DMA,
            pltpu.SemaphoreType.DMA,
            pltpu.SemaphoreType.DMA,
            pltpu.SemaphoreType.DMA,
            pltpu.SemaphoreType.DMA,
            pltpu.VMEM_SHARED((N,), jnp.int32),
            pltpu.VMEM_SHARED((N,), jnp.int32),
            pltpu.VMEM_SHARED((NT, 256), jnp.int32),
        ],
    )
    return f(scores, g, h)


def kernel(g, h, proj_w, proj_b):
    scores = _scores(h, proj_w, proj_b)
    g_new, new_h, idx = _sc_fused(scores, g, h)
    return g_new, new_h, idx


# speedup vs baseline: 4.1396x; 1.0917x over previous
import functools

import jax
import jax.numpy as jnp
from jax import lax
from jax.experimental import pallas as pl
from jax.experimental.pallas import tpu as pltpu
from jax.experimental.pallas import tpu_sc as plsc

N = 8192
D = 256
KN = 4096
L = 16
NW = 32
RPW = KN // NW
CR = 2
NCH = RPW // CR
HC = 32
NCV = KN // L
NDV = D // L


def _scores_body(h_ref, w_ref, b_ref, out_ref):
    acc = jnp.dot(h_ref[...], w_ref[...])
    out_ref[...] = jax.nn.sigmoid(acc + b_ref[0, 0])


def _scores(h, proj_w, proj_b):
    b2 = proj_b.reshape(1, 1)
    out = pl.pallas_call(
        _scores_body,
        out_shape=jax.ShapeDtypeStruct((N, 1), jnp.float32),
        in_specs=[
            pl.BlockSpec((N, D), lambda: (0, 0)),
            pl.BlockSpec((D, 1), lambda: (0, 0)),
            pl.BlockSpec(memory_space=pltpu.SMEM),
        ],
        out_specs=pl.BlockSpec((N, 1), lambda: (0, 0)),
    )(h, proj_w, b2)
    return out[:, 0]


NT = 16
EPT = N // NT
EV = EPT // L
NPASS = 4
OCH = KN // NT


def _sc_fused_body(scores_hbm, g_hbm, h_hbm,
                   gout_hbm, hout_hbm, idx_hbm,
                   svals, keys, pay, histL, histT, allhist, tot, bbase, cnt,
                   ksc, psc, stg_k, stg_p, stg_pos, pout, kb,
                   vidx, rowpad, vvals, grow, gout, hbuf,
                   gsem0, gsem1, osem00, osem01, osem10, osem11, hsem, hsem2,
                   skeys_sh, spay_sh, hist_sh):
    cid = lax.axis_index("c")
    tid = lax.axis_index("s")
    base = tid * EPT
    iota = lax.iota(jnp.int32, L)
    zero_i = jnp.zeros((L,), jnp.int32)

    pltpu.sync_copy(scores_hbm.at[pl.ds(base, EPT)], svals)

    @plsc.parallel_loop(0, EV, 1, unroll=4)
    def _(v):
        off = pl.multiple_of(v * L, L)
        s = svals[pl.ds(off, L)]
        bu = lax.bitcast_convert_type(s, jnp.uint32)
        neg = (bu >> 31) == jnp.uint32(1)
        m = jnp.where(neg, ~bu, bu | jnp.uint32(0x80000000))
        keys[pl.ds(off, L)] = lax.bitcast_convert_type(~m, jnp.int32)
        pay[pl.ds(off, L)] = base + v * L + iota

    for p in range(NPASS):
        sh = 8 * p

        @plsc.parallel_loop(0, 16 * 256 // L, 1, unroll=8)
        def _(bz):
            off = pl.multiple_of(bz * L, L)
            histL[pl.ds(off, L)] = zero_i

        def dohist(v, _):
            off = pl.multiple_of(v * L, L)
            k = keys[pl.ds(off, L)]
            d = (lax.shift_right_logical(k, sh)) & 255
            plsc.addupdate_scatter(histL, [d * L + iota],
                                   jnp.ones((L,), jnp.int32))
            return 0

        lax.fori_loop(0, EV, dohist, 0)

        @plsc.parallel_loop(0, 256, 1, unroll=4)
        def _(bh):
            off = pl.multiple_of(bh * L, L)
            histL[pl.ds(off, L)] = plsc.cumsum(histL[pl.ds(off, L)])

        @plsc.parallel_loop(0, 256 // L, 1, unroll=4)
        def _(bv):
            off = pl.multiple_of(bv * L, L)
            idxs = (bv * L + iota) * L + (L - 1)
            histT[pl.ds(off, L)] = plsc.load_gather(histL, [idxs])

        pltpu.sync_copy(histT, hist_sh.at[tid])
        plsc.subcore_barrier()

        pltpu.sync_copy(hist_sh, allhist)

        @plsc.parallel_loop(0, 256 // L, 1, unroll=2)
        def _(bv):
            off = pl.multiple_of(bv * L, L)

            def acc_t(t, carry):
                tv, mv = carry
                row = allhist[t, pl.ds(off, L)]
                tv = tv + row
                mv = mv + jnp.where(t < tid, row, zero_i)
                return (tv, mv)

            tv, mv = lax.fori_loop(0, NT, acc_t, (zero_i, zero_i))
            tot[pl.ds(off, L)] = tv
            bbase[pl.ds(off, L)] = mv

        def excl_bv(bv, carry):
            off = pl.multiple_of(bv * L, L)
            tv = tot[pl.ds(off, L)]
            inc = plsc.cumsum(tv)
            bbase[pl.ds(off, L)] = bbase[pl.ds(off, L)] + (inc - tv) + carry
            return carry + jnp.sum(tv)

        lax.fori_loop(0, 256 // L, excl_bv, jnp.int32(0))

        def zcnt(bv, _):
            off = pl.multiple_of(bv * L, L)
            cnt[pl.ds(off, L)] = zero_i
            return 0

        lax.fori_loop(0, 256 // L, zcnt, 0)

        def permute(v, _):
            off = pl.multiple_of(v * L, L)
            k = keys[pl.ds(off, L)]
            pv = pay[pl.ds(off, L)]
            d = (lax.shift_right_logical(k, sh)) & 255
            d_s, lane_s = lax.sort([d, iota], dimension=0, is_stable=True,
                                   num_keys=1)
            ksc[...] = d_s
            prev = plsc.load_gather(ksc, [jnp.maximum(iota - 1, 0)])
            prev = jnp.where(iota == 0, jnp.full((L,), -1, jnp.int32), prev)
            seg = d_s != prev
            first = plsc.cummax(jnp.where(seg, iota, zero_i))
            subr = iota - first
            nxt = plsc.load_gather(ksc, [jnp.minimum(iota + 1, L - 1)])
            nxt = jnp.where(iota == L - 1, jnp.full((L,), -1, jnp.int32), nxt)
            last = d_s != nxt
            cnt_old = plsc.load_gather(cnt, [d_s])
            bb = plsc.load_gather(bbase, [d_s])
            pos = bb + cnt_old + subr
            plsc.store_scatter(cnt, [d_s], cnt_old + subr + 1, mask=last)
            ksc[...] = k
            psc[...] = pv
            k_s = plsc.load_gather(ksc, [lane_s])
            p_s = plsc.load_gather(psc, [lane_s])
            row = v >> 3
            col = pl.multiple_of((v & 7) * L, L)
            stg_k[row, pl.ds(col, L)] = k_s
            stg_p[row, pl.ds(col, L)] = p_s
            stg_pos[row, pl.ds(col, L)] = pos
            return 0

        lax.fori_loop(0, EV, permute, 0)

        for j in range(4):
            pltpu.sync_copy(stg_k.at[j], skeys_sh.at[stg_pos.at[j]])
            pltpu.sync_copy(stg_p.at[j], spay_sh.at[stg_pos.at[j]])
        plsc.subcore_barrier()

        if p < NPASS - 1:
            pltpu.sync_copy(skeys_sh.at[pl.ds(base, EPT)], keys)
            pltpu.sync_copy(spay_sh.at[pl.ds(base, EPT)], pay)

    @pl.when(cid == 0)
    def _():
        obase = tid * OCH
        pltpu.sync_copy(spay_sh.at[pl.ds(obase, OCH)], pout)
        pltpu.sync_copy(pout, idx_hbm.at[pl.ds(obase, OCH)])

    sid = tid
    wid = sid * 2 + cid
    base = wid * RPW

    pltpu.sync_copy(spay_sh.at[pl.ds(0, KN)], vidx)
    pltpu.sync_copy(skeys_sh.at[pl.ds(base, RPW)], kb)

    def unkey(ov, _):
        off = pl.multiple_of(ov * L, L)
        m = ~lax.bitcast_convert_type(kb[pl.ds(off, L)], jnp.uint32)
        negflag = (m >> 31) == jnp.uint32(1)
        bu = jnp.where(negflag, m & jnp.uint32(0x7FFFFFFF), ~m)
        vvals[pl.ds(off, L)] = lax.bitcast_convert_type(bu, jnp.float32)
        return 0

    lax.fori_loop(0, RPW // L, unkey, 0)

    for v in range(NCH * 8 // L):
        p = lax.iota(jnp.int32, L) + v * L
        src = (p >> 3) * CR + jnp.minimum(p & 7, CR - 1)
        rv = plsc.load_gather(vidx, [src + base])
        rowpad[pl.ds(v * L, L)] = rv

    gsems = (gsem0, gsem1)
    osems = ((osem00, osem01), (osem10, osem11))

    for b in range(2):
        pltpu.async_copy(g_hbm.at[rowpad.at[pl.ds(8 * b, CR)]],
                         grow.at[b], gsems[b])

    NHC = RPW // HC
    hsems = (hsem, hsem2)
    for hb in range(2):
        pltpu.async_copy(h_hbm.at[vidx.at[pl.ds(base + hb * HC, HC)]],
                         hbuf.at[hb], hsems[hb])
    for hc in range(NHC):
        hb = hc & 1
        pltpu.make_async_copy(h_hbm.at[pl.ds(0, HC)], hbuf.at[hb],
                              hsems[hb]).wait()

        def hrow(r, _):
            sval = plsc.load_gather(
                vvals, [jnp.full((L,), hc * HC, jnp.int32) + r])

            def hcv(cv, _):
                off = pl.multiple_of(cv * L, L)
                hbuf[hb, r, pl.ds(off, L)] = hbuf[hb, r, pl.ds(off, L)] * sval
                return 0

            return lax.fori_loop(0, NDV, hcv, 0)

        lax.fori_loop(0, HC, hrow, 0)
        pltpu.sync_copy(hbuf.at[hb], hout_hbm.at[pl.ds(base + hc * HC, HC)])
        if hc + 2 < NHC:
            pltpu.async_copy(
                h_hbm.at[vidx.at[pl.ds(base + (hc + 2) * HC, HC)]],
                hbuf.at[hb], hsems[hb])

    def chunk(c, b):
        pltpu.make_async_copy(g_hbm.at[pl.ds(0, CR)], grow.at[b],
                              gsems[b]).wait()
        zero = jnp.zeros((L,), jnp.float32)
        bidx = jnp.full((L,), b, jnp.int32)
        r0i = jnp.full((L,), 0, jnp.int32)
        r1i = jnp.full((L,), 1, jnp.int32)

        gb = (b * CR + 0) * KN
        gb1 = (b * CR + 1) * KN

        @plsc.parallel_loop(0, NCV, 1, unroll=8, carry=(zero, zero))
        def accs(cv, carry):
            a0, a1 = carry
            off = pl.multiple_of(cv * L, L)
            ci = vidx[pl.ds(off, L)]
            v0 = plsc.load_gather(grow, [bidx, r0i, ci])
            v1 = plsc.load_gather(grow, [bidx, r1i, ci])
            gout[pl.ds(gb + off, L)] = v0
            gout[pl.ds(gb1 + off, L)] = v1
            return (a0 + v0, a1 + v1)

        scales = []
        for r in range(CR):
            tot_v = jnp.full((L,), jnp.sum(accs[r]), jnp.float32)
            scales.append(jnp.ones((L,), jnp.float32) / (tot_v + 1e-9))

            @pl.when(c >= 2)
            def _():
                pltpu.make_async_copy(gout.at[pl.ds(0, KN)],
                                      gout_hbm.at[0], osems[b][r]).wait()

        @plsc.parallel_loop(0, NCV, 1, unroll=8)
        def _(cv):
            off = pl.multiple_of(cv * L, L)
            gout[pl.ds(gb + off, L)] = gout[pl.ds(gb + off, L)] * scales[0]
            gout[pl.ds(gb1 + off, L)] = gout[pl.ds(gb1 + off, L)] * scales[1]

        for r in range(CR):
            rg = base + c * CR + r
            pltpu.async_copy(gout.at[pl.ds((b * CR + r) * KN, KN)],
                             gout_hbm.at[rg], osems[b][r])

        @pl.when(c + 2 < NCH)
        def _():
            off = pl.multiple_of(8 * (c + 2), 8)
            pltpu.async_copy(
                g_hbm.at[rowpad.at[pl.ds(off, CR)]],
                grow.at[b], gsems[b])

    def outer(i, _):
        for b in range(2):
            chunk(i * 2 + b, b)
        return 0

    lax.fori_loop(0, NCH // 2, outer, 0)

    for b in range(2):
        for r in range(CR):
            pltpu.make_async_copy(gout.at[pl.ds(0, KN)], gout_hbm.at[0],
                                  osems[b][r]).wait()


def _sc_fused(scores, g, h):
    mesh = plsc.VectorSubcoreMesh(core_axis_name="c", subcore_axis_name="s")
    f = pl.kernel(
        _sc_fused_body,
        out_type=(
            jax.ShapeDtypeStruct((KN, KN), jnp.float32),
            jax.ShapeDtypeStruct((KN, D), jnp.float32),
            jax.ShapeDtypeStruct((KN,), jnp.int32),
        ),
        mesh=mesh,
        compiler_params=pltpu.CompilerParams(needs_layout_passes=False),
        scratch_types=[
            pltpu.VMEM((EPT,), jnp.float32),
            pltpu.VMEM((EPT,), jnp.int32),
            pltpu.VMEM((EPT,), jnp.int32),
            pltpu.VMEM((256 * L,), jnp.int32),
            pltpu.VMEM((256,), jnp.int32),
            pltpu.VMEM((NT, 256), jnp.int32),
            pltpu.VMEM((256,), jnp.int32),
            pltpu.VMEM((256,), jnp.int32),
            pltpu.VMEM((256,), jnp.int32),
            pltpu.VMEM((L,), jnp.int32),
            pltpu.VMEM((L,), jnp.int32),
            pltpu.VMEM((4, 128), jnp.int32),
            pltpu.VMEM((4, 128), jnp.int32),
            pltpu.VMEM((4, 128), jnp.int32),
            pltpu.VMEM((OCH,), jnp.int32),
            pltpu.VMEM((RPW,), jnp.int32),
            pltpu.VMEM((KN,), jnp.int32),
            pltpu.VMEM((NCH * 8,), jnp.int32),
            pltpu.VMEM((RPW,), jnp.float32),
            pltpu.VMEM((2, CR, N), jnp.float32),
            pltpu.VMEM((2 * CR * KN,), jnp.float32),
            pltpu.VMEM((2, HC, D), jnp.float32),
            pltpu.SemaphoreType.DMA,
            pltpu.SemaphoreType.DMA,
            pltpu.SemaphoreType.DMA,
            pltpu.SemaphoreType.DMA,
            pltpu.SemaphoreType.DMA,
            pltpu.SemaphoreType.DMA,
            pltpu.SemaphoreType.DMA,
            pltpu.SemaphoreType.DMA,
            pltpu.VMEM_SHARED((N,), jnp.int32),
            pltpu.VMEM_SHARED((N,), jnp.int32),
            pltpu.VMEM_SHARED((NT, 256), jnp.int32),
        ],
    )
    return f(scores, g, h)


def kernel(g, h, proj_w, proj_b):
    scores = _scores(h, proj_w, proj_b)
    g_new, new_h, idx = _sc_fused(scores, g, h)
    return g_new, new_h, idx


# speedup vs baseline: 4.1767x; 1.0090x over previous
import functools

import jax
import jax.numpy as jnp
from jax import lax
from jax.experimental import pallas as pl
from jax.experimental.pallas import tpu as pltpu
from jax.experimental.pallas import tpu_sc as plsc

N = 8192
D = 256
KN = 4096
L = 16
NW = 32
RPW = KN // NW
CR = 2
NCH = RPW // CR
HC = 32
NCV = KN // L
NDV = D // L


def _scores_body(h_ref, w_ref, b_ref, out_ref):
    acc = jnp.dot(h_ref[...], w_ref[...])
    out_ref[...] = jax.nn.sigmoid(acc + b_ref[0, 0])


def _scores(h, proj_w, proj_b):
    b2 = proj_b.reshape(1, 1)
    out = pl.pallas_call(
        _scores_body,
        out_shape=jax.ShapeDtypeStruct((N, 1), jnp.float32),
        in_specs=[
            pl.BlockSpec((N, D), lambda: (0, 0)),
            pl.BlockSpec((D, 1), lambda: (0, 0)),
            pl.BlockSpec(memory_space=pltpu.SMEM),
        ],
        out_specs=pl.BlockSpec((N, 1), lambda: (0, 0)),
    )(h, proj_w, b2)
    return out[:, 0]


NT = 16
EPT = N // NT
EV = EPT // L
NPASS = 4
OCH = KN // NT


def _sc_fused_body(scores_hbm, g_hbm, h_hbm,
                   gout_hbm, hout_hbm, idx_hbm,
                   svals, keys, pay, histL, histT, allhist, tot, bbase, cnt,
                   dsc_all, kall, pall, spp, slst, sinc,
                   stg_k, stg_p, stg_pos, pout, kb,
                   vidx, rowpad, vvals, grow, gout, hbuf,
                   gsem0, gsem1, osem00, osem01, osem10, osem11, hsem, hsem2,
                   skeys_sh, spay_sh, hist_sh):
    cid = lax.axis_index("c")
    tid = lax.axis_index("s")
    base = tid * EPT
    iota = lax.iota(jnp.int32, L)
    zero_i = jnp.zeros((L,), jnp.int32)

    pltpu.sync_copy(scores_hbm.at[pl.ds(base, EPT)], svals)

    @plsc.parallel_loop(0, EV, 1, unroll=4)
    def _(v):
        off = pl.multiple_of(v * L, L)
        s = svals[pl.ds(off, L)]
        bu = lax.bitcast_convert_type(s, jnp.uint32)
        neg = (bu >> 31) == jnp.uint32(1)
        m = jnp.where(neg, ~bu, bu | jnp.uint32(0x80000000))
        keys[pl.ds(off, L)] = lax.bitcast_convert_type(~m, jnp.int32)
        pay[pl.ds(off, L)] = base + v * L + iota

    for p in range(NPASS):
        sh = 8 * p

        @plsc.parallel_loop(0, 16 * 256 // L, 1, unroll=8)
        def _(bz):
            off = pl.multiple_of(bz * L, L)
            histL[pl.ds(off, L)] = zero_i

        @plsc.parallel_loop(0, EV, 1, unroll=4)
        def _(v):
            off = pl.multiple_of(v * L, L)
            k = keys[pl.ds(off, L)]
            d = (lax.shift_right_logical(k, sh)) & 255
            plsc.addupdate_scatter(histL, [d * L + iota],
                                   jnp.ones((L,), jnp.int32))

        @plsc.parallel_loop(0, 256, 1, unroll=4)
        def _(bh):
            off = pl.multiple_of(bh * L, L)
            histL[pl.ds(off, L)] = plsc.cumsum(histL[pl.ds(off, L)])

        @plsc.parallel_loop(0, 256 // L, 1, unroll=4)
        def _(bv):
            off = pl.multiple_of(bv * L, L)
            idxs = (bv * L + iota) * L + (L - 1)
            histT[pl.ds(off, L)] = plsc.load_gather(histL, [idxs])

        pltpu.sync_copy(histT, hist_sh.at[tid])
        plsc.subcore_barrier()

        pltpu.sync_copy(hist_sh, allhist)

        @plsc.parallel_loop(0, 256 // L, 1, unroll=2)
        def _(bv):
            off = pl.multiple_of(bv * L, L)

            def acc_t(t, carry):
                tv, mv = carry
                row = allhist[t, pl.ds(off, L)]
                tv = tv + row
                mv = mv + jnp.where(t < tid, row, zero_i)
                return (tv, mv)

            tv, mv = lax.fori_loop(0, NT, acc_t, (zero_i, zero_i))
            tot[pl.ds(off, L)] = tv
            bbase[pl.ds(off, L)] = mv

        def excl_bv(bv, carry):
            off = pl.multiple_of(bv * L, L)
            tv = tot[pl.ds(off, L)]
            inc = plsc.cumsum(tv)
            bbase[pl.ds(off, L)] = bbase[pl.ds(off, L)] + (inc - tv) + carry
            return carry + jnp.sum(tv)

        lax.fori_loop(0, 256 // L, excl_bv, jnp.int32(0))

        def zcnt(bv, _):
            off = pl.multiple_of(bv * L, L)
            cnt[pl.ds(off, L)] = zero_i
            return 0

        lax.fori_loop(0, 256 // L, zcnt, 0)

        @plsc.parallel_loop(0, EV, 1, unroll=2)
        def _(v):
            off = pl.multiple_of(v * L, L)
            k = keys[pl.ds(off, L)]
            pv = pay[pl.ds(off, L)]
            d = (lax.shift_right_logical(k, sh)) & 255
            d_s, lane_s = lax.sort([d, iota], dimension=0, is_stable=True,
                                   num_keys=1)
            dsc_all[pl.ds(off, L)] = d_s
            prev = plsc.load_gather(dsc_all, [off + jnp.maximum(iota - 1, 0)])
            prev = jnp.where(iota == 0, jnp.full((L,), -1, jnp.int32), prev)
            seg = d_s != prev
            first = plsc.cummax(jnp.where(seg, iota, zero_i))
            subr = iota - first
            nxt = plsc.load_gather(dsc_all,
                                   [off + jnp.minimum(iota + 1, L - 1)])
            nxt = jnp.where(iota == L - 1, jnp.full((L,), -1, jnp.int32), nxt)
            last = d_s != nxt
            bb = plsc.load_gather(bbase, [d_s])
            kall[pl.ds(off, L)] = k
            pall[pl.ds(off, L)] = pv
            k_s = plsc.load_gather(kall, [off + lane_s])
            p_s = plsc.load_gather(pall, [off + lane_s])
            row = v >> 3
            col = pl.multiple_of((v & 7) * L, L)
            stg_k[row, pl.ds(col, L)] = k_s
            stg_p[row, pl.ds(col, L)] = p_s
            spp[pl.ds(off, L)] = bb + subr
            slst[pl.ds(off, L)] = jnp.where(last, iota * 0 + 1, zero_i)
            sinc[pl.ds(off, L)] = subr + 1

        def permute2(v, _):
            off = pl.multiple_of(v * L, L)
            d_s = dsc_all[pl.ds(off, L)]
            cnt_old = plsc.load_gather(cnt, [d_s])
            pos = spp[pl.ds(off, L)] + cnt_old
            row = v >> 3
            col = pl.multiple_of((v & 7) * L, L)
            stg_pos[row, pl.ds(col, L)] = pos
            lastm = slst[pl.ds(off, L)] != 0
            plsc.store_scatter(cnt, [d_s], sinc[pl.ds(off, L)] + cnt_old,
                               mask=lastm)
            return 0

        lax.fori_loop(0, EV, permute2, 0)

        for j in range(4):
            pltpu.sync_copy(stg_k.at[j], skeys_sh.at[stg_pos.at[j]])
            pltpu.sync_copy(stg_p.at[j], spay_sh.at[stg_pos.at[j]])
        plsc.subcore_barrier()

        if p < NPASS - 1:
            pltpu.sync_copy(skeys_sh.at[pl.ds(base, EPT)], keys)
            pltpu.sync_copy(spay_sh.at[pl.ds(base, EPT)], pay)

    @pl.when(cid == 0)
    def _():
        obase = tid * OCH
        pltpu.sync_copy(spay_sh.at[pl.ds(obase, OCH)], pout)
        pltpu.sync_copy(pout, idx_hbm.at[pl.ds(obase, OCH)])

    sid = tid
    wid = sid * 2 + cid
    base = wid * RPW

    pltpu.sync_copy(spay_sh.at[pl.ds(0, KN)], vidx)
    pltpu.sync_copy(skeys_sh.at[pl.ds(base, RPW)], kb)

    def unkey(ov, _):
        off = pl.multiple_of(ov * L, L)
        m = ~lax.bitcast_convert_type(kb[pl.ds(off, L)], jnp.uint32)
        negflag = (m >> 31) == jnp.uint32(1)
        bu = jnp.where(negflag, m & jnp.uint32(0x7FFFFFFF), ~m)
        vvals[pl.ds(off, L)] = lax.bitcast_convert_type(bu, jnp.float32)
        return 0

    lax.fori_loop(0, RPW // L, unkey, 0)

    for v in range(NCH * 8 // L):
        p = lax.iota(jnp.int32, L) + v * L
        src = (p >> 3) * CR + jnp.minimum(p & 7, CR - 1)
        rv = plsc.load_gather(vidx, [src + base])
        rowpad[pl.ds(v * L, L)] = rv

    gsems = (gsem0, gsem1)
    osems = ((osem00, osem01), (osem10, osem11))

    for b in range(2):
        pltpu.async_copy(g_hbm.at[rowpad.at[pl.ds(8 * b, CR)]],
                         grow.at[b], gsems[b])

    NHC = RPW // HC
    hsems = (hsem, hsem2)
    for hb in range(2):
        pltpu.async_copy(h_hbm.at[vidx.at[pl.ds(base + hb * HC, HC)]],
                         hbuf.at[hb], hsems[hb])
    for hc in range(NHC):
        hb = hc & 1
        pltpu.make_async_copy(h_hbm.at[pl.ds(0, HC)], hbuf.at[hb],
                              hsems[hb]).wait()

        def hrow(r, _):
            sval = plsc.load_gather(
                vvals, [jnp.full((L,), hc * HC, jnp.int32) + r])

            def hcv(cv, _):
                off = pl.multiple_of(cv * L, L)
                hbuf[hb, r, pl.ds(off, L)] = hbuf[hb, r, pl.ds(off, L)] * sval
                return 0

            return lax.fori_loop(0, NDV, hcv, 0)

        lax.fori_loop(0, HC, hrow, 0)
        pltpu.sync_copy(hbuf.at[hb], hout_hbm.at[pl.ds(base + hc * HC, HC)])
        if hc + 2 < NHC:
            pltpu.async_copy(
                h_hbm.at[vidx.at[pl.ds(base + (hc + 2) * HC, HC)]],
                hbuf.at[hb], hsems[hb])

    def chunk(c, b):
        pltpu.make_async_copy(g_hbm.at[pl.ds(0, CR)], grow.at[b],
                              gsems[b]).wait()
        zero = jnp.zeros((L,), jnp.float32)
        bidx = jnp.full((L,), b, jnp.int32)
        r0i = jnp.full((L,), 0, jnp.int32)
        r1i = jnp.full((L,), 1, jnp.int32)

        gb = (b * CR + 0) * KN
        gb1 = (b * CR + 1) * KN

        @plsc.parallel_loop(0, NCV, 1, unroll=8, carry=(zero, zero))
        def accs(cv, carry):
            a0, a1 = carry
            off = pl.multiple_of(cv * L, L)
            ci = vidx[pl.ds(off, L)]
            v0 = plsc.load_gather(grow, [bidx, r0i, ci])
            v1 = plsc.load_gather(grow, [bidx, r1i, ci])
            gout[pl.ds(gb + off, L)] = v0
            gout[pl.ds(gb1 + off, L)] = v1
            return (a0 + v0, a1 + v1)

        scales = []
        for r in range(CR):
            tot_v = jnp.full((L,), jnp.sum(accs[r]), jnp.float32)
            scales.append(jnp.ones((L,), jnp.float32) / (tot_v + 1e-9))

            @pl.when(c >= 2)
            def _():
                pltpu.make_async_copy(gout.at[pl.ds(0, KN)],
                                      gout_hbm.at[0], osems[b][r]).wait()

        @plsc.parallel_loop(0, NCV, 1, unroll=8)
        def _(cv):
            off = pl.multiple_of(cv * L, L)
            gout[pl.ds(gb + off, L)] = gout[pl.ds(gb + off, L)] * scales[0]
            gout[pl.ds(gb1 + off, L)] = gout[pl.ds(gb1 + off, L)] * scales[1]

        for r in range(CR):
            rg = base + c * CR + r
            pltpu.async_copy(gout.at[pl.ds((b * CR + r) * KN, KN)],
                             gout_hbm.at[rg], osems[b][r])

        @pl.when(c + 2 < NCH)
        def _():
            off = pl.multiple_of(8 * (c + 2), 8)
            pltpu.async_copy(
                g_hbm.at[rowpad.at[pl.ds(off, CR)]],
                grow.at[b], gsems[b])

    def outer(i, _):
        for b in range(2):
            chunk(i * 2 + b, b)
        return 0

    lax.fori_loop(0, NCH // 2, outer, 0)

    for b in range(2):
        for r in range(CR):
            pltpu.make_async_copy(gout.at[pl.ds(0, KN)], gout_hbm.at[0],
                                  osems[b][r]).wait()


def _sc_fused(scores, g, h):
    mesh = plsc.VectorSubcoreMesh(core_axis_name="c", subcore_axis_name="s")
    f = pl.kernel(
        _sc_fused_body,
        out_type=(
            jax.ShapeDtypeStruct((KN, KN), jnp.float32),
            jax.ShapeDtypeStruct((KN, D), jnp.float32),
            jax.ShapeDtypeStruct((KN,), jnp.int32),
        ),
        mesh=mesh,
        compiler_params=pltpu.CompilerParams(needs_layout_passes=False),
        scratch_types=[
            pltpu.VMEM((EPT,), jnp.float32),
            pltpu.VMEM((EPT,), jnp.int32),
            pltpu.VMEM((EPT,), jnp.int32),
            pltpu.VMEM((256 * L,), jnp.int32),
            pltpu.VMEM((256,), jnp.int32),
            pltpu.VMEM((NT, 256), jnp.int32),
            pltpu.VMEM((256,), jnp.int32),
            pltpu.VMEM((256,), jnp.int32),
            pltpu.VMEM((256,), jnp.int32),
            pltpu.VMEM((EPT,), jnp.int32),
            pltpu.VMEM((EPT,), jnp.int32),
            pltpu.VMEM((EPT,), jnp.int32),
            pltpu.VMEM((EPT,), jnp.int32),
            pltpu.VMEM((EPT,), jnp.int32),
            pltpu.VMEM((EPT,), jnp.int32),
            pltpu.VMEM((4, 128), jnp.int32),
            pltpu.VMEM((4, 128), jnp.int32),
            pltpu.VMEM((4, 128), jnp.int32),
            pltpu.VMEM((OCH,), jnp.int32),
            pltpu.VMEM((RPW,), jnp.int32),
            pltpu.VMEM((KN,), jnp.int32),
            pltpu.VMEM((NCH * 8,), jnp.int32),
            pltpu.VMEM((RPW,), jnp.float32),
            pltpu.VMEM((2, CR, N), jnp.float32),
            pltpu.VMEM((2 * CR * KN,), jnp.float32),
            pltpu.VMEM((2, HC, D), jnp.float32),
            pltpu.SemaphoreType.DMA,
            pltpu.SemaphoreType.DMA,
            pltpu.SemaphoreType.DMA,
            pltpu.SemaphoreType.DMA,
            pltpu.SemaphoreType.DMA,
            pltpu.SemaphoreType.DMA,
            pltpu.SemaphoreType.DMA,
            pltpu.SemaphoreType.DMA,
            pltpu.VMEM_SHARED((N,), jnp.int32),
            pltpu.VMEM_SHARED((N,), jnp.int32),
            pltpu.VMEM_SHARED((NT, 256), jnp.int32),
        ],
    )
    return f(scores, g, h)


def kernel(g, h, proj_w, proj_b):
    scores = _scores(h, proj_w, proj_b)
    g_new, new_h, idx = _sc_fused(scores, g, h)
    return g_new, new_h, idx


# speedup vs baseline: 4.2447x; 1.0163x over previous
import functools

import jax
import jax.numpy as jnp
from jax import lax
from jax.experimental import pallas as pl
from jax.experimental.pallas import tpu as pltpu
from jax.experimental.pallas import tpu_sc as plsc

N = 8192
D = 256
KN = 4096
L = 16
NW = 32
RPW = KN // NW
CR = 2
NCH = RPW // CR
HC = 32
NCV = KN // L
NDV = D // L


def _scores_body(h_ref, w_ref, b_ref, out_ref):
    acc = jnp.dot(h_ref[...], w_ref[...])
    out_ref[...] = jax.nn.sigmoid(acc + b_ref[0, 0])


def _scores(h, proj_w, proj_b):
    b2 = proj_b.reshape(1, 1)
    out = pl.pallas_call(
        _scores_body,
        out_shape=jax.ShapeDtypeStruct((N, 1), jnp.float32),
        in_specs=[
            pl.BlockSpec((N, D), lambda: (0, 0)),
            pl.BlockSpec((D, 1), lambda: (0, 0)),
            pl.BlockSpec(memory_space=pltpu.SMEM),
        ],
        out_specs=pl.BlockSpec((N, 1), lambda: (0, 0)),
    )(h, proj_w, b2)
    return out[:, 0]


NT = 16
EPT = N // NT
EV = EPT // L
NPASS = 4
OCH = KN // NT


def _sc_fused_body(scores_hbm, g_hbm, h_hbm,
                   gout_hbm, hout_hbm, idx_hbm,
                   svals, keys, pay, histL, histT, allhist, tot, bbase, cnt,
                   dsc_all, kall, pall, spp, slst, sinc,
                   stg_k, stg_p, stg_pos, pout, kb, scsem,
                   vidx, rowpad, vvals, grow, gout, hbuf,
                   gsem0, gsem1, osem00, osem01, osem10, osem11, hsem, hsem2,
                   skeys_sh, spay_sh, hist_sh):
    cid = lax.axis_index("c")
    tid = lax.axis_index("s")
    base = tid * EPT
    iota = lax.iota(jnp.int32, L)
    zero_i = jnp.zeros((L,), jnp.int32)

    pltpu.sync_copy(scores_hbm.at[pl.ds(base, EPT)], svals)

    @plsc.parallel_loop(0, EV, 1, unroll=4)
    def _(v):
        off = pl.multiple_of(v * L, L)
        s = svals[pl.ds(off, L)]
        bu = lax.bitcast_convert_type(s, jnp.uint32)
        neg = (bu >> 31) == jnp.uint32(1)
        m = jnp.where(neg, ~bu, bu | jnp.uint32(0x80000000))
        keys[pl.ds(off, L)] = lax.bitcast_convert_type(~m, jnp.int32)
        pay[pl.ds(off, L)] = base + v * L + iota

    for p in range(NPASS):
        sh = 8 * p

        @plsc.parallel_loop(0, 16 * 256 // L, 1, unroll=8)
        def _(bz):
            off = pl.multiple_of(bz * L, L)
            histL[pl.ds(off, L)] = zero_i

        @plsc.parallel_loop(0, EV, 1, unroll=4)
        def _(v):
            off = pl.multiple_of(v * L, L)
            k = keys[pl.ds(off, L)]
            d = (lax.shift_right_logical(k, sh)) & 255
            plsc.addupdate_scatter(histL, [d * L + iota],
                                   jnp.ones((L,), jnp.int32))

        @plsc.parallel_loop(0, 256, 1, unroll=4)
        def _(bh):
            off = pl.multiple_of(bh * L, L)
            histL[pl.ds(off, L)] = plsc.cumsum(histL[pl.ds(off, L)])

        @plsc.parallel_loop(0, 256 // L, 1, unroll=4)
        def _(bv):
            off = pl.multiple_of(bv * L, L)
            idxs = (bv * L + iota) * L + (L - 1)
            histT[pl.ds(off, L)] = plsc.load_gather(histL, [idxs])

        pltpu.sync_copy(histT, hist_sh.at[tid])
        plsc.subcore_barrier()

        pltpu.sync_copy(hist_sh, allhist)

        @plsc.parallel_loop(0, 256 // L, 1, unroll=2)
        def _(bv):
            off = pl.multiple_of(bv * L, L)

            def acc_t(t, carry):
                tv, mv = carry
                row = allhist[t, pl.ds(off, L)]
                tv = tv + row
                mv = mv + jnp.where(t < tid, row, zero_i)
                return (tv, mv)

            tv, mv = lax.fori_loop(0, NT, acc_t, (zero_i, zero_i))
            tot[pl.ds(off, L)] = tv
            bbase[pl.ds(off, L)] = mv

        def excl_bv(bv, carry):
            off = pl.multiple_of(bv * L, L)
            tv = tot[pl.ds(off, L)]
            inc = plsc.cumsum(tv)
            bbase[pl.ds(off, L)] = bbase[pl.ds(off, L)] + (inc - tv) + carry
            return carry + jnp.sum(tv)

        lax.fori_loop(0, 256 // L, excl_bv, jnp.int32(0))

        def zcnt(bv, _):
            off = pl.multiple_of(bv * L, L)
            cnt[pl.ds(off, L)] = zero_i
            return 0

        lax.fori_loop(0, 256 // L, zcnt, 0)

        @plsc.parallel_loop(0, EV, 1, unroll=2)
        def _(v):
            off = pl.multiple_of(v * L, L)
            k = keys[pl.ds(off, L)]
            pv = pay[pl.ds(off, L)]
            d = (lax.shift_right_logical(k, sh)) & 255
            d_s, lane_s = lax.sort([d, iota], dimension=0, is_stable=True,
                                   num_keys=1)
            dsc_all[pl.ds(off, L)] = d_s
            prev = plsc.load_gather(dsc_all, [off + jnp.maximum(iota - 1, 0)])
            prev = jnp.where(iota == 0, jnp.full((L,), -1, jnp.int32), prev)
            seg = d_s != prev
            first = plsc.cummax(jnp.where(seg, iota, zero_i))
            subr = iota - first
            nxt = plsc.load_gather(dsc_all,
                                   [off + jnp.minimum(iota + 1, L - 1)])
            nxt = jnp.where(iota == L - 1, jnp.full((L,), -1, jnp.int32), nxt)
            last = d_s != nxt
            bb = plsc.load_gather(bbase, [d_s])
            kall[pl.ds(off, L)] = k
            pall[pl.ds(off, L)] = pv
            k_s = plsc.load_gather(kall, [off + lane_s])
            p_s = plsc.load_gather(pall, [off + lane_s])
            row = v >> 3
            col = pl.multiple_of((v & 7) * L, L)
            stg_k[row, pl.ds(col, L)] = k_s
            stg_p[row, pl.ds(col, L)] = p_s
            spp[pl.ds(off, L)] = bb + subr
            slst[pl.ds(off, L)] = jnp.where(last, iota * 0 + 1, zero_i)
            sinc[pl.ds(off, L)] = subr + 1

        def permute2(v, _):
            off = pl.multiple_of(v * L, L)
            d_s = dsc_all[pl.ds(off, L)]
            cnt_old = plsc.load_gather(cnt, [d_s])
            pos = spp[pl.ds(off, L)] + cnt_old
            row = v >> 3
            col = pl.multiple_of((v & 7) * L, L)
            stg_pos[row, pl.ds(col, L)] = pos
            lastm = slst[pl.ds(off, L)] != 0
            plsc.store_scatter(cnt, [d_s], sinc[pl.ds(off, L)] + cnt_old,
                               mask=lastm)
            return 0

        lax.fori_loop(0, EV, permute2, 0)

        for j in range(4):
            pltpu.async_copy(stg_k.at[j], skeys_sh.at[stg_pos.at[j]], scsem)
            pltpu.async_copy(stg_p.at[j], spay_sh.at[stg_pos.at[j]], scsem)
        for j in range(4):
            pltpu.make_async_copy(stg_k.at[j], skeys_sh.at[stg_pos.at[j]],
                                  scsem).wait()
            pltpu.make_async_copy(stg_p.at[j], spay_sh.at[stg_pos.at[j]],
                                  scsem).wait()
        plsc.subcore_barrier()

        if p < NPASS - 1:
            pltpu.sync_copy(skeys_sh.at[pl.ds(base, EPT)], keys)
            pltpu.sync_copy(spay_sh.at[pl.ds(base, EPT)], pay)

    @pl.when(cid == 0)
    def _():
        obase = tid * OCH
        pltpu.sync_copy(spay_sh.at[pl.ds(obase, OCH)], pout)
        pltpu.sync_copy(pout, idx_hbm.at[pl.ds(obase, OCH)])

    sid = tid
    wid = sid * 2 + cid
    base = wid * RPW

    pltpu.sync_copy(spay_sh.at[pl.ds(0, KN)], vidx)
    pltpu.sync_copy(skeys_sh.at[pl.ds(base, RPW)], kb)

    def unkey(ov, _):
        off = pl.multiple_of(ov * L, L)
        m = ~lax.bitcast_convert_type(kb[pl.ds(off, L)], jnp.uint32)
        negflag = (m >> 31) == jnp.uint32(1)
        bu = jnp.where(negflag, m & jnp.uint32(0x7FFFFFFF), ~m)
        vvals[pl.ds(off, L)] = lax.bitcast_convert_type(bu, jnp.float32)
        return 0

    lax.fori_loop(0, RPW // L, unkey, 0)

    for v in range(NCH * 8 // L):
        p = lax.iota(jnp.int32, L) + v * L
        src = (p >> 3) * CR + jnp.minimum(p & 7, CR - 1)
        rv = plsc.load_gather(vidx, [src + base])
        rowpad[pl.ds(v * L, L)] = rv

    gsems = (gsem0, gsem1)
    osems = ((osem00, osem01), (osem10, osem11))

    for b in range(2):
        pltpu.async_copy(g_hbm.at[rowpad.at[pl.ds(8 * b, CR)]],
                         grow.at[b], gsems[b])

    NHC = RPW // HC
    hsems = (hsem, hsem2)
    for hb in range(2):
        pltpu.async_copy(h_hbm.at[vidx.at[pl.ds(base + hb * HC, HC)]],
                         hbuf.at[hb], hsems[hb])
    for hc in range(NHC):
        hb = hc & 1
        pltpu.make_async_copy(h_hbm.at[pl.ds(0, HC)], hbuf.at[hb],
                              hsems[hb]).wait()

        def hrow(r, _):
            sval = plsc.load_gather(
                vvals, [jnp.full((L,), hc * HC, jnp.int32) + r])

            def hcv(cv, _):
                off = pl.multiple_of(cv * L, L)
                hbuf[hb, r, pl.ds(off, L)] = hbuf[hb, r, pl.ds(off, L)] * sval
                return 0

            return lax.fori_loop(0, NDV, hcv, 0)

        lax.fori_loop(0, HC, hrow, 0)
        pltpu.sync_copy(hbuf.at[hb], hout_hbm.at[pl.ds(base + hc * HC, HC)])
        if hc + 2 < NHC:
            pltpu.async_copy(
                h_hbm.at[vidx.at[pl.ds(base + (hc + 2) * HC, HC)]],
                hbuf.at[hb], hsems[hb])

    def chunk(c, b):
        pltpu.make_async_copy(g_hbm.at[pl.ds(0, CR)], grow.at[b],
                              gsems[b]).wait()
        zero = jnp.zeros((L,), jnp.float32)
        bidx = jnp.full((L,), b, jnp.int32)
        r0i = jnp.full((L,), 0, jnp.int32)
        r1i = jnp.full((L,), 1, jnp.int32)

        gb = (b * CR + 0) * KN
        gb1 = (b * CR + 1) * KN

        @plsc.parallel_loop(0, NCV, 1, unroll=8, carry=(zero, zero))
        def accs(cv, carry):
            a0, a1 = carry
            off = pl.multiple_of(cv * L, L)
            ci = vidx[pl.ds(off, L)]
            v0 = plsc.load_gather(grow, [bidx, r0i, ci])
            v1 = plsc.load_gather(grow, [bidx, r1i, ci])
            gout[pl.ds(gb + off, L)] = v0
            gout[pl.ds(gb1 + off, L)] = v1
            return (a0 + v0, a1 + v1)

        scales = []
        for r in range(CR):
            tot_v = jnp.full((L,), jnp.sum(accs[r]), jnp.float32)
            scales.append(jnp.ones((L,), jnp.float32) / (tot_v + 1e-9))

            @pl.when(c >= 2)
            def _():
                pltpu.make_async_copy(gout.at[pl.ds(0, KN)],
                                      gout_hbm.at[0], osems[b][r]).wait()

        @plsc.parallel_loop(0, NCV, 1, unroll=8)
        def _(cv):
            off = pl.multiple_of(cv * L, L)
            gout[pl.ds(gb + off, L)] = gout[pl.ds(gb + off, L)] * scales[0]
            gout[pl.ds(gb1 + off, L)] = gout[pl.ds(gb1 + off, L)] * scales[1]

        for r in range(CR):
            rg = base + c * CR + r
            pltpu.async_copy(gout.at[pl.ds((b * CR + r) * KN, KN)],
                             gout_hbm.at[rg], osems[b][r])

        @pl.when(c + 2 < NCH)
        def _():
            off = pl.multiple_of(8 * (c + 2), 8)
            pltpu.async_copy(
                g_hbm.at[rowpad.at[pl.ds(off, CR)]],
                grow.at[b], gsems[b])

    def outer(i, _):
        for b in range(2):
            chunk(i * 2 + b, b)
        return 0

    lax.fori_loop(0, NCH // 2, outer, 0)

    for b in range(2):
        for r in range(CR):
            pltpu.make_async_copy(gout.at[pl.ds(0, KN)], gout_hbm.at[0],
                                  osems[b][r]).wait()


def _sc_fused(scores, g, h):
    mesh = plsc.VectorSubcoreMesh(core_axis_name="c", subcore_axis_name="s")
    f = pl.kernel(
        _sc_fused_body,
        out_type=(
            jax.ShapeDtypeStruct((KN, KN), jnp.float32),
            jax.ShapeDtypeStruct((KN, D), jnp.float32),
            jax.ShapeDtypeStruct((KN,), jnp.int32),
        ),
        mesh=mesh,
        compiler_params=pltpu.CompilerParams(needs_layout_passes=False),
        scratch_types=[
            pltpu.VMEM((EPT,), jnp.float32),
            pltpu.VMEM((EPT,), jnp.int32),
            pltpu.VMEM((EPT,), jnp.int32),
            pltpu.VMEM((256 * L,), jnp.int32),
            pltpu.VMEM((256,), jnp.int32),
            pltpu.VMEM((NT, 256), jnp.int32),
            pltpu.VMEM((256,), jnp.int32),
            pltpu.VMEM((256,), jnp.int32),
            pltpu.VMEM((256,), jnp.int32),
            pltpu.VMEM((EPT,), jnp.int32),
            pltpu.VMEM((EPT,), jnp.int32),
            pltpu.VMEM((EPT,), jnp.int32),
            pltpu.VMEM((EPT,), jnp.int32),
            pltpu.VMEM((EPT,), jnp.int32),
            pltpu.VMEM((EPT,), jnp.int32),
            pltpu.VMEM((4, 128), jnp.int32),
            pltpu.VMEM((4, 128), jnp.int32),
            pltpu.VMEM((4, 128), jnp.int32),
            pltpu.VMEM((OCH,), jnp.int32),
            pltpu.VMEM((RPW,), jnp.int32),
            pltpu.SemaphoreType.DMA,
            pltpu.VMEM((KN,), jnp.int32),
            pltpu.VMEM((NCH * 8,), jnp.int32),
            pltpu.VMEM((RPW,), jnp.float32),
            pltpu.VMEM((2, CR, N), jnp.float32),
            pltpu.VMEM((2 * CR * KN,), jnp.float32),
            pltpu.VMEM((2, HC, D), jnp.float32),
            pltpu.SemaphoreType.DMA,
            pltpu.SemaphoreType.DMA,
            pltpu.SemaphoreType.DMA,
            pltpu.SemaphoreType.DMA,
            pltpu.SemaphoreType.DMA,
            pltpu.SemaphoreType.DMA,
            pltpu.SemaphoreType.DMA,
            pltpu.SemaphoreType.DMA,
            pltpu.VMEM_SHARED((N,), jnp.int32),
            pltpu.VMEM_SHARED((N,), jnp.int32),
            pltpu.VMEM_SHARED((NT, 256), jnp.int32),
        ],
    )
    return f(scores, g, h)


def kernel(g, h, proj_w, proj_b):
    scores = _scores(h, proj_w, proj_b)
    g_new, new_h, idx = _sc_fused(scores, g, h)
    return g_new, new_h, idx
